# baseline (speedup 1.0000x reference)
import functools

import jax
import jax.numpy as jnp
from jax import lax
from jax.experimental import pallas as pl
from jax.experimental.pallas import tpu as pltpu
from jax.experimental.pallas import tpu_sc as plsc

B = 16384
D = 16
N_BIG = 1000000
N_USR = 100000
NREL = 64
CF_LAMBDA = 1e-05
KG_LAMBDA = 1e-05

_NC, _NS = 2, 16
NW = _NC * _NS
BPW = B // NW
EPW = BPW * D


@functools.cache
def _make_sc_gather():
    mesh = plsc.VectorSubcoreMesh(core_axis_name="c", subcore_axis_name="s")

    @functools.partial(
        pl.kernel,
        mesh=mesh,
        out_type=[jax.ShapeDtypeStruct((B * D,), jnp.float32)] * 8,
        scratch_types=[
            pltpu.VMEM((EPW,), jnp.int32),
            pltpu.VMEM((EPW,), jnp.float32),
            pltpu.SemaphoreType.DMA,
        ],
    )
    def _sc_gather(user_flat, item_flat, entity_flat,
                   off_u, off_ip, off_ineg, off_h, off_pt, off_nt,
                   u_out, ip_out, ineg_out, ipk_out, inegk_out,
                   he_out, pt_out, nt_out,
                   idx_v, rows_v, sem):
        wid = lax.axis_index("s") * _NC + lax.axis_index("c")
        base = wid * EPW

        def load_off(off):
            pltpu.sync_copy(off.at[pl.ds(base, EPW)], idx_v)

        def gather_to(tab, out):
            pltpu.async_copy(tab.at[idx_v], rows_v, sem).wait()
            pltpu.sync_copy(rows_v, out.at[pl.ds(base, EPW)])

        load_off(off_u)
        gather_to(user_flat, u_out)
        load_off(off_ip)
        gather_to(item_flat, ip_out)
        gather_to(entity_flat, ipk_out)
        load_off(off_ineg)
        gather_to(item_flat, ineg_out)
        gather_to(entity_flat, inegk_out)
        load_off(off_h)
        gather_to(entity_flat, he_out)
        load_off(off_pt)
        gather_to(entity_flat, pt_out)
        load_off(off_nt)
        gather_to(entity_flat, nt_out)

    return _sc_gather


def _tc_body(u_ref, ip_ref, ineg_ref, ipk_ref, inegk_ref,
             he_ref, pt_ref, nt_ref, r_ref, relW_ref, M2d_ref, out_ref):
    i = pl.program_id(0)

    @pl.when(i == 0)
    def _init():
        out_ref[...] = jnp.zeros((1, 1), jnp.float32)

    u = u_ref[...]
    pos_cf = ip_ref[...] + ipk_ref[...]
    neg_cf = ineg_ref[...] + inegk_ref[...]
    pos_s = jnp.sum(u * pos_cf, axis=1, keepdims=True)
    neg_s = jnp.sum(u * neg_cf, axis=1, keepdims=True)
    x = pos_s - neg_s
    sig = 1.0 / (1.0 + jnp.exp(-x))
    cf_term = -jnp.log(1e-10 + sig)
    l2_cf = 0.5 * (jnp.sum(u * u) + jnp.sum(pos_cf * pos_cf)
                   + jnp.sum(neg_cf * neg_cf))

    r_col = r_ref[...]
    k_row = lax.broadcasted_iota(jnp.int32, (1, NREL), 1)
    onehot = (r_col == k_row).astype(jnp.float32)
    Weff = jnp.dot(onehot, M2d_ref[...],
                   preferred_element_type=jnp.float32)
    re = jnp.dot(onehot, relW_ref[...],
                 preferred_element_type=jnp.float32)

    c1 = lax.broadcasted_iota(jnp.int32, (D, D * D), 1)
    d1 = lax.broadcasted_iota(jnp.int32, (D, D * D), 0)
    R = ((c1 // D) == d1).astype(jnp.float32)
    c2 = lax.broadcasted_iota(jnp.int32, (D * D, D), 0)
    j2 = lax.broadcasted_iota(jnp.int32, (D * D, D), 1)
    S = ((c2 % D) == j2).astype(jnp.float32)

    def proj(x16):
        xr = jnp.dot(x16, R, preferred_element_type=jnp.float32)
        return jnp.dot(xr * Weff, S, preferred_element_type=jnp.float32)

    rh = proj(he_ref[...])
    rpt = proj(pt_ref[...])
    rnt = proj(nt_ref[...])

    def normz(v):
        n = jnp.sqrt(jnp.sum(v * v, axis=1, keepdims=True))
        return v / jnp.maximum(n, 1e-12)

    re_n = normz(re)
    rh_n = normz(rh)
    rpt_n = normz(rpt)
    rnt_n = normz(rnt)

    dpos = rh_n + re_n - rpt_n
    dneg = rh_n + re_n - rnt_n
    pos_sc = jnp.sqrt(jnp.sum(dpos * dpos, axis=1, keepdims=True))
    neg_sc = jnp.sqrt(jnp.sum(dneg * dneg, axis=1, keepdims=True))
    kg_term = jnp.maximum(pos_sc - neg_sc + 1.0, 0.0)
    l2_kg = 0.5 * (jnp.sum(rh_n * rh_n) + jnp.sum(re_n * re_n)
                   + jnp.sum(rpt_n * rpt_n) + jnp.sum(rnt_n * rnt_n))

    block_total = (jnp.sum(cf_term) + CF_LAMBDA * l2_cf
                   + jnp.sum(kg_term) + KG_LAMBDA * l2_kg)
    out_ref[...] += jnp.reshape(block_total * (1.0 / B), (1, 1))


_BB = 2048


def _tc_call(gathered, r2, rel_W, M2d, interpret=False):
    row_spec = pl.BlockSpec((_BB, D), lambda i: (i, 0))
    out = pl.pallas_call(
        _tc_body,
        grid=(B // _BB,),
        in_specs=[row_spec] * 8 + [
            pl.BlockSpec((_BB, 1), lambda i: (i, 0)),
            pl.BlockSpec((NREL, D), lambda i: (0, 0)),
            pl.BlockSpec((NREL, D * D), lambda i: (0, 0)),
        ],
        out_specs=pl.BlockSpec((1, 1), lambda i: (0, 0)),
        out_shape=jax.ShapeDtypeStruct((1, 1), jnp.float32),
        interpret=interpret,
    )(*gathered, r2, rel_W, M2d)
    return out[0, 0]


def _offsets(idx, n_rows):
    k = jnp.arange(B * D, dtype=jnp.int32)
    return (k % D) * n_rows + jnp.repeat(idx, D)


def kernel(user_W, item_W, entity_W, rel_W, trans_M,
           user_ids, item_pos_ids, item_neg_ids, h, r, pos_t, neg_t,
           is_train=1):
    i32 = lambda a: a.astype(jnp.int32)
    uid, ipid, inegid = i32(user_ids), i32(item_pos_ids), i32(item_neg_ids)
    hid, ptid, ntid = i32(h), i32(pos_t), i32(neg_t)

    user_flat = user_W.T.reshape(-1)
    item_flat = item_W.T.reshape(-1)
    entity_flat = entity_W.T.reshape(-1)

    offs = [_offsets(uid, N_USR)] + [
        _offsets(a, N_BIG) for a in (ipid, inegid, hid, ptid, ntid)]

    flat_out = _make_sc_gather()(user_flat, item_flat, entity_flat, *offs)
    gathered = [f.reshape(B, D) for f in flat_out]

    M2d = trans_M.reshape(NREL, D * D)
    r2 = i32(r).reshape(B, 1)
    return _tc_call(gathered, r2, rel_W, M2d)


# speedup vs baseline: 6.5866x; 6.5866x over previous
import functools

import jax
import jax.numpy as jnp
from jax import lax
from jax.experimental import pallas as pl
from jax.experimental.pallas import tpu as pltpu
from jax.experimental.pallas import tpu_sc as plsc

B = 16384
D = 16
N_BIG = 1000000
N_USR = 100000
NREL = 64
CF_LAMBDA = 1e-05
KG_LAMBDA = 1e-05

_NC, _NS = 2, 16
NW = _NC * _NS
BPW = B // NW
EPW = BPW * D


_CW = 16384


@functools.cache
def _make_sc_detile():
    mesh = plsc.VectorSubcoreMesh(core_axis_name="c", subcore_axis_name="s")

    @functools.partial(
        pl.kernel,
        mesh=mesh,
        out_type=[
            jax.ShapeDtypeStruct((D * N_USR,), jnp.float32),
            jax.ShapeDtypeStruct((D * N_BIG,), jnp.float32),
            jax.ShapeDtypeStruct((D * N_BIG,), jnp.float32),
        ],
        scratch_types=[
            pltpu.VMEM((_CW,), jnp.float32),
            pltpu.VMEM((64,), jnp.float32),
        ],
    )
    def _sc_detile(userT, itemT, entityT, user_flat, item_flat, entity_flat,
                   vbuf, zbuf):
        wid = lax.axis_index("s") * _NC + lax.axis_index("c")

        for i0 in range(0, 64, 16):
            zbuf[pl.ds(i0, 16)] = jnp.zeros((16,), jnp.float32)

        def chunk_copy(tabT, flat, n_rows, j, c):
            c0 = c * _CW
            pltpu.sync_copy(tabT.at[j, pl.ds(c0, _CW)], vbuf)
            pltpu.sync_copy(vbuf, flat.at[pl.ds(j * n_rows + c0, _CW)])

        n_full_big = 999424 // _CW
        n_full_usr = 98304 // _CW
        for j in range(D):
            for rep in range(2):
                c = wid + 32 * rep

                @pl.when(c < n_full_big)
                def _():
                    chunk_copy(itemT, item_flat, N_BIG, j, c)
                    chunk_copy(entityT, entity_flat, N_BIG, j, c)

            @pl.when(wid < n_full_usr)
            def _():
                chunk_copy(userT, user_flat, N_USR, j, wid)

            @pl.when(wid == j)
            def _():
                pltpu.sync_copy(itemT.at[j, pl.ds(999424, 512)],
                                vbuf.at[pl.ds(0, 512)])
                pltpu.sync_copy(vbuf.at[pl.ds(0, 512)],
                                item_flat.at[pl.ds(j * N_BIG + 999424, 512)])
                pltpu.sync_copy(userT.at[j, pl.ds(98304, 1664)],
                                vbuf.at[pl.ds(0, 1664)])
                pltpu.sync_copy(vbuf.at[pl.ds(0, 1664)],
                                user_flat.at[pl.ds(j * N_USR + 98304, 1664)])
                pltpu.sync_copy(zbuf.at[pl.ds(0, 32)],
                                user_flat.at[pl.ds(j * N_USR + 99968, 32)])

            @pl.when(wid == D + j)
            def _():
                pltpu.sync_copy(entityT.at[j, pl.ds(999424, 512)],
                                vbuf.at[pl.ds(0, 512)])
                pltpu.sync_copy(vbuf.at[pl.ds(0, 512)],
                                entity_flat.at[pl.ds(j * N_BIG + 999424, 512)])
                pltpu.sync_copy(zbuf,
                                item_flat.at[pl.ds(j * N_BIG + 999936, 64)])
                pltpu.sync_copy(zbuf,
                                entity_flat.at[pl.ds(j * N_BIG + 999936, 64)])

    return _sc_detile


@functools.cache
def _make_sc_gather():
    mesh = plsc.VectorSubcoreMesh(core_axis_name="c", subcore_axis_name="s")

    @functools.partial(
        pl.kernel,
        mesh=mesh,
        out_type=[jax.ShapeDtypeStruct((B * D,), jnp.float32)] * 8,
        scratch_types=[
            pltpu.VMEM((EPW,), jnp.int32),
            pltpu.VMEM((EPW,), jnp.float32),
            pltpu.SemaphoreType.DMA,
        ],
    )
    def _sc_gather(user_flat, item_flat, entity_flat,
                   off_u, off_ip, off_ineg, off_h, off_pt, off_nt,
                   u_out, ip_out, ineg_out, ipk_out, inegk_out,
                   he_out, pt_out, nt_out,
                   idx_v, rows_v, sem):
        wid = lax.axis_index("s") * _NC + lax.axis_index("c")
        base = wid * EPW

        def load_off(off):
            pltpu.sync_copy(off.at[pl.ds(base, EPW)], idx_v)

        def gather_to(tab, out):
            pltpu.async_copy(tab.at[idx_v], rows_v, sem).wait()
            pltpu.sync_copy(rows_v, out.at[pl.ds(base, EPW)])

        load_off(off_u)
        gather_to(user_flat, u_out)
        load_off(off_ip)
        gather_to(item_flat, ip_out)
        gather_to(entity_flat, ipk_out)
        load_off(off_ineg)
        gather_to(item_flat, ineg_out)
        gather_to(entity_flat, inegk_out)
        load_off(off_h)
        gather_to(entity_flat, he_out)
        load_off(off_pt)
        gather_to(entity_flat, pt_out)
        load_off(off_nt)
        gather_to(entity_flat, nt_out)

    return _sc_gather


N_USR_VALID = 99968
N_BIG_VALID = 999936


def _patch(x, id_col, n_valid, tail_tab, width):
    oh = (id_col - n_valid == lax.broadcasted_iota(jnp.int32, (1, width), 1))
    patched = jnp.dot(oh.astype(jnp.float32), tail_tab,
                      preferred_element_type=jnp.float32)
    return jnp.where(id_col >= n_valid, patched, x)


def _tc_body(u_ref, ip_ref, ineg_ref, ipk_ref, inegk_ref,
             he_ref, pt_ref, nt_ref,
             uid_ref, ipid_ref, inegid_ref, hid_ref, ptid_ref, ntid_ref,
             r_ref, relW_ref, M2d_ref,
             tailu_ref, taili_ref, taile_ref, out_ref):
    i = pl.program_id(0)

    @pl.when(i == 0)
    def _init():
        out_ref[...] = jnp.zeros((1, 1), jnp.float32)

    tailu = tailu_ref[...]
    taili = taili_ref[...]
    taile = taile_ref[...]
    uid = uid_ref[...]
    ipid = ipid_ref[...]
    inegid = inegid_ref[...]

    u = _patch(u_ref[...], uid, N_USR_VALID, tailu, 32)
    ip = _patch(ip_ref[...], ipid, N_BIG_VALID, taili, 64)
    ipk = _patch(ipk_ref[...], ipid, N_BIG_VALID, taile, 64)
    ineg = _patch(ineg_ref[...], inegid, N_BIG_VALID, taili, 64)
    inegk = _patch(inegk_ref[...], inegid, N_BIG_VALID, taile, 64)
    pos_cf = ip + ipk
    neg_cf = ineg + inegk
    pos_s = jnp.sum(u * pos_cf, axis=1, keepdims=True)
    neg_s = jnp.sum(u * neg_cf, axis=1, keepdims=True)
    x = pos_s - neg_s
    sig = 1.0 / (1.0 + jnp.exp(-x))
    cf_term = -jnp.log(1e-10 + sig)
    l2_cf = 0.5 * (jnp.sum(u * u) + jnp.sum(pos_cf * pos_cf)
                   + jnp.sum(neg_cf * neg_cf))

    r_col = r_ref[...]
    k_row = lax.broadcasted_iota(jnp.int32, (1, NREL), 1)
    onehot = (r_col == k_row).astype(jnp.float32)
    Weff = jnp.dot(onehot, M2d_ref[...],
                   preferred_element_type=jnp.float32)
    re = jnp.dot(onehot, relW_ref[...],
                 preferred_element_type=jnp.float32)

    c1 = lax.broadcasted_iota(jnp.int32, (D, D * D), 1)
    d1 = lax.broadcasted_iota(jnp.int32, (D, D * D), 0)
    R = ((c1 // D) == d1).astype(jnp.float32)
    c2 = lax.broadcasted_iota(jnp.int32, (D * D, D), 0)
    j2 = lax.broadcasted_iota(jnp.int32, (D * D, D), 1)
    S = ((c2 % D) == j2).astype(jnp.float32)

    def proj(x16):
        xr = jnp.dot(x16, R, preferred_element_type=jnp.float32)
        return jnp.dot(xr * Weff, S, preferred_element_type=jnp.float32)

    rh = proj(_patch(he_ref[...], hid_ref[...], N_BIG_VALID, taile, 64))
    rpt = proj(_patch(pt_ref[...], ptid_ref[...], N_BIG_VALID, taile, 64))
    rnt = proj(_patch(nt_ref[...], ntid_ref[...], N_BIG_VALID, taile, 64))

    def normz(v):
        n = jnp.sqrt(jnp.sum(v * v, axis=1, keepdims=True))
        return v / jnp.maximum(n, 1e-12)

    re_n = normz(re)
    rh_n = normz(rh)
    rpt_n = normz(rpt)
    rnt_n = normz(rnt)

    dpos = rh_n + re_n - rpt_n
    dneg = rh_n + re_n - rnt_n
    pos_sc = jnp.sqrt(jnp.sum(dpos * dpos, axis=1, keepdims=True))
    neg_sc = jnp.sqrt(jnp.sum(dneg * dneg, axis=1, keepdims=True))
    kg_term = jnp.maximum(pos_sc - neg_sc + 1.0, 0.0)
    l2_kg = 0.5 * (jnp.sum(rh_n * rh_n) + jnp.sum(re_n * re_n)
                   + jnp.sum(rpt_n * rpt_n) + jnp.sum(rnt_n * rnt_n))

    block_total = (jnp.sum(cf_term) + CF_LAMBDA * l2_cf
                   + jnp.sum(kg_term) + KG_LAMBDA * l2_kg)
    out_ref[...] += jnp.reshape(block_total * (1.0 / B), (1, 1))


_BB = 2048


def _tc_call(gathered, id_cols, r2, rel_W, M2d, tails, interpret=False):
    row_spec = pl.BlockSpec((_BB, D), lambda i: (i, 0))
    col_spec = pl.BlockSpec((_BB, 1), lambda i: (i, 0))
    out = pl.pallas_call(
        _tc_body,
        grid=(B // _BB,),
        in_specs=[row_spec] * 8 + [col_spec] * 7 + [
            pl.BlockSpec((NREL, D), lambda i: (0, 0)),
            pl.BlockSpec((NREL, D * D), lambda i: (0, 0)),
            pl.BlockSpec((32, D), lambda i: (0, 0)),
            pl.BlockSpec((64, D), lambda i: (0, 0)),
            pl.BlockSpec((64, D), lambda i: (0, 0)),
        ],
        out_specs=pl.BlockSpec((1, 1), lambda i: (0, 0)),
        out_shape=jax.ShapeDtypeStruct((1, 1), jnp.float32),
        interpret=interpret,
    )(*gathered, *id_cols, r2, rel_W, M2d, *tails)
    return out[0, 0]


def _offsets(idx, n_rows):
    k = jnp.arange(B * D, dtype=jnp.int32)
    return (k % D) * n_rows + jnp.repeat(idx, D)


def kernel(user_W, item_W, entity_W, rel_W, trans_M,
           user_ids, item_pos_ids, item_neg_ids, h, r, pos_t, neg_t,
           is_train=1):
    i32 = lambda a: a.astype(jnp.int32)
    uid, ipid, inegid = i32(user_ids), i32(item_pos_ids), i32(item_neg_ids)
    hid, ptid, ntid = i32(h), i32(pos_t), i32(neg_t)

    user_flat, item_flat, entity_flat = _make_sc_detile()(
        user_W.T, item_W.T, entity_W.T)

    offs = [_offsets(uid, N_USR)] + [
        _offsets(a, N_BIG) for a in (ipid, inegid, hid, ptid, ntid)]

    flat_out = _make_sc_gather()(user_flat, item_flat, entity_flat, *offs)
    gathered = [f.reshape(B, D) for f in flat_out]

    id_cols = [a.reshape(B, 1) for a in (uid, ipid, inegid, hid, ptid, ntid)]
    tails = [user_W[N_USR_VALID:], item_W[N_BIG_VALID:], entity_W[N_BIG_VALID:]]
    M2d = trans_M.reshape(NREL, D * D)
    r2 = i32(r).reshape(B, 1)
    return _tc_call(gathered, id_cols, r2, rel_W, M2d, tails)


# speedup vs baseline: 7.0347x; 1.0680x over previous
import functools

import jax
import jax.numpy as jnp
from jax import lax
from jax.experimental import pallas as pl
from jax.experimental.pallas import tpu as pltpu
from jax.experimental.pallas import tpu_sc as plsc

B = 16384
D = 16
N_BIG = 1000000
N_USR = 100000
NREL = 64
CF_LAMBDA = 1e-05
KG_LAMBDA = 1e-05

_NC, _NS = 2, 16
NW = _NC * _NS
BPW = B // NW
EPW = BPW * D


_CW = 16384


@functools.cache
def _make_sc_detile():
    mesh = plsc.VectorSubcoreMesh(core_axis_name="c", subcore_axis_name="s")

    @functools.partial(
        pl.kernel,
        mesh=mesh,
        out_type=[
            jax.ShapeDtypeStruct((D * N_USR,), jnp.float32),
            jax.ShapeDtypeStruct((D * N_BIG,), jnp.float32),
            jax.ShapeDtypeStruct((D * N_BIG,), jnp.float32),
        ],
        scratch_types=[
            pltpu.VMEM((_CW,), jnp.float32),
            pltpu.VMEM((64,), jnp.float32),
        ],
    )
    def _sc_detile(userT, itemT, entityT, user_flat, item_flat, entity_flat,
                   vbuf, zbuf):
        wid = lax.axis_index("s") * _NC + lax.axis_index("c")

        for i0 in range(0, 64, 16):
            zbuf[pl.ds(i0, 16)] = jnp.zeros((16,), jnp.float32)

        def chunk_copy(tabT, flat, n_rows, j, c):
            c0 = c * _CW
            pltpu.sync_copy(tabT.at[j, pl.ds(c0, _CW)], vbuf)
            pltpu.sync_copy(vbuf, flat.at[pl.ds(j * n_rows + c0, _CW)])

        n_full_big = 999424 // _CW
        n_full_usr = 98304 // _CW
        for j in range(D):
            for rep in range(2):
                c = wid + 32 * rep

                @pl.when(c < n_full_big)
                def _():
                    chunk_copy(itemT, item_flat, N_BIG, j, c)
                    chunk_copy(entityT, entity_flat, N_BIG, j, c)

            @pl.when(wid < n_full_usr)
            def _():
                chunk_copy(userT, user_flat, N_USR, j, wid)

            @pl.when(wid == j)
            def _():
                pltpu.sync_copy(itemT.at[j, pl.ds(999424, 512)],
                                vbuf.at[pl.ds(0, 512)])
                pltpu.sync_copy(vbuf.at[pl.ds(0, 512)],
                                item_flat.at[pl.ds(j * N_BIG + 999424, 512)])
                pltpu.sync_copy(userT.at[j, pl.ds(98304, 1664)],
                                vbuf.at[pl.ds(0, 1664)])
                pltpu.sync_copy(vbuf.at[pl.ds(0, 1664)],
                                user_flat.at[pl.ds(j * N_USR + 98304, 1664)])
                pltpu.sync_copy(zbuf.at[pl.ds(0, 32)],
                                user_flat.at[pl.ds(j * N_USR + 99968, 32)])

            @pl.when(wid == D + j)
            def _():
                pltpu.sync_copy(entityT.at[j, pl.ds(999424, 512)],
                                vbuf.at[pl.ds(0, 512)])
                pltpu.sync_copy(vbuf.at[pl.ds(0, 512)],
                                entity_flat.at[pl.ds(j * N_BIG + 999424, 512)])
                pltpu.sync_copy(zbuf,
                                item_flat.at[pl.ds(j * N_BIG + 999936, 64)])
                pltpu.sync_copy(zbuf,
                                entity_flat.at[pl.ds(j * N_BIG + 999936, 64)])

    return _sc_detile


@functools.cache
def _make_sc_gather():
    mesh = plsc.VectorSubcoreMesh(core_axis_name="c", subcore_axis_name="s")

    @functools.partial(
        pl.kernel,
        mesh=mesh,
        out_type=[jax.ShapeDtypeStruct((B * D,), jnp.float32)] * 8,
        scratch_types=[
            pltpu.VMEM((EPW,), jnp.int32),
            pltpu.VMEM((EPW,), jnp.float32),
            pltpu.SemaphoreType.DMA,
        ],
    )
    def _sc_gather(user_flat, item_flat, entity_flat,
                   off_u, off_ip, off_ineg, off_h, off_pt, off_nt,
                   u_out, ip_out, ineg_out, ipk_out, inegk_out,
                   he_out, pt_out, nt_out,
                   idx_v, rows_v, sem):
        wid = lax.axis_index("s") * _NC + lax.axis_index("c")
        base = wid * EPW

        def load_off(off):
            pltpu.sync_copy(off.at[pl.ds(base, EPW)], idx_v)

        def gather_to(tab, out):
            pltpu.async_copy(tab.at[idx_v], rows_v, sem).wait()
            pltpu.sync_copy(rows_v, out.at[pl.ds(base, EPW)])

        load_off(off_u)
        gather_to(user_flat, u_out)
        load_off(off_ip)
        gather_to(item_flat, ip_out)
        gather_to(entity_flat, ipk_out)
        load_off(off_ineg)
        gather_to(item_flat, ineg_out)
        gather_to(entity_flat, inegk_out)
        load_off(off_h)
        gather_to(entity_flat, he_out)
        load_off(off_pt)
        gather_to(entity_flat, pt_out)
        load_off(off_nt)
        gather_to(entity_flat, nt_out)

    return _sc_gather


N_USR_VALID = 99968
N_BIG_VALID = 999936


def _patch(x, id_col, n_valid, tail_tab, width):
    oh = (id_col - n_valid == lax.broadcasted_iota(jnp.int32, (1, width), 1))
    patched = jnp.dot(oh.astype(jnp.float32), tail_tab,
                      preferred_element_type=jnp.float32)
    return jnp.where(id_col >= n_valid, patched, x)


def _tc_body(u_ref, ip_ref, ineg_ref, ipk_ref, inegk_ref,
             he_ref, pt_ref, nt_ref,
             uid_ref, ipid_ref, inegid_ref, hid_ref, ptid_ref, ntid_ref,
             r_ref, relW_ref, M2d_ref,
             tailu_ref, taili_ref, taile_ref, out_ref):
    i = pl.program_id(0)

    @pl.when(i == 0)
    def _init():
        out_ref[...] = jnp.zeros((1, 1), jnp.float32)

    tailu = tailu_ref[...]
    taili = taili_ref[...]
    taile = taile_ref[...]
    uid = uid_ref[...]
    ipid = ipid_ref[...]
    inegid = inegid_ref[...]

    def unpack(ref):
        x = ref[...]
        return jnp.concatenate(
            [x[:, D * s:D * (s + 1)] for s in range(8)], axis=0)

    u = _patch(unpack(u_ref), uid, N_USR_VALID, tailu, 32)
    ip = _patch(unpack(ip_ref), ipid, N_BIG_VALID, taili, 64)
    ipk = _patch(unpack(ipk_ref), ipid, N_BIG_VALID, taile, 64)
    ineg = _patch(unpack(ineg_ref), inegid, N_BIG_VALID, taili, 64)
    inegk = _patch(unpack(inegk_ref), inegid, N_BIG_VALID, taile, 64)
    pos_cf = ip + ipk
    neg_cf = ineg + inegk
    pos_s = jnp.sum(u * pos_cf, axis=1, keepdims=True)
    neg_s = jnp.sum(u * neg_cf, axis=1, keepdims=True)
    x = pos_s - neg_s
    sig = 1.0 / (1.0 + jnp.exp(-x))
    cf_term = -jnp.log(1e-10 + sig)
    l2_cf = 0.5 * (jnp.sum(u * u) + jnp.sum(pos_cf * pos_cf)
                   + jnp.sum(neg_cf * neg_cf))

    r_col = r_ref[...]
    k_row = lax.broadcasted_iota(jnp.int32, (1, NREL), 1)
    onehot = (r_col == k_row).astype(jnp.float32)
    Weff = jnp.dot(onehot, M2d_ref[...],
                   preferred_element_type=jnp.float32)
    re = jnp.dot(onehot, relW_ref[...],
                 preferred_element_type=jnp.float32)

    c1 = lax.broadcasted_iota(jnp.int32, (D, D * D), 1)
    d1 = lax.broadcasted_iota(jnp.int32, (D, D * D), 0)
    R = ((c1 // D) == d1).astype(jnp.float32)
    c2 = lax.broadcasted_iota(jnp.int32, (D * D, D), 0)
    j2 = lax.broadcasted_iota(jnp.int32, (D * D, D), 1)
    S = ((c2 % D) == j2).astype(jnp.float32)

    def proj(x16):
        xr = jnp.dot(x16, R, preferred_element_type=jnp.float32)
        return jnp.dot(xr * Weff, S, preferred_element_type=jnp.float32)

    rh = proj(_patch(unpack(he_ref), hid_ref[...], N_BIG_VALID, taile, 64))
    rpt = proj(_patch(unpack(pt_ref), ptid_ref[...], N_BIG_VALID, taile, 64))
    rnt = proj(_patch(unpack(nt_ref), ntid_ref[...], N_BIG_VALID, taile, 64))

    def normz(v):
        n = jnp.sqrt(jnp.sum(v * v, axis=1, keepdims=True))
        return v / jnp.maximum(n, 1e-12)

    re_n = normz(re)
    rh_n = normz(rh)
    rpt_n = normz(rpt)
    rnt_n = normz(rnt)

    dpos = rh_n + re_n - rpt_n
    dneg = rh_n + re_n - rnt_n
    pos_sc = jnp.sqrt(jnp.sum(dpos * dpos, axis=1, keepdims=True))
    neg_sc = jnp.sqrt(jnp.sum(dneg * dneg, axis=1, keepdims=True))
    kg_term = jnp.maximum(pos_sc - neg_sc + 1.0, 0.0)
    l2_kg = 0.5 * (jnp.sum(rh_n * rh_n) + jnp.sum(re_n * re_n)
                   + jnp.sum(rpt_n * rpt_n) + jnp.sum(rnt_n * rnt_n))

    block_total = (jnp.sum(cf_term) + CF_LAMBDA * l2_cf
                   + jnp.sum(kg_term) + KG_LAMBDA * l2_kg)
    out_ref[...] += jnp.reshape(block_total * (1.0 / B), (1, 1))


_BB = 2048


def _tc_call(gathered, id_cols, r2, rel_W, M2d, tails, interpret=False):
    row_spec = pl.BlockSpec((_BB // 8, 128), lambda i: (i, 0))
    col_spec = pl.BlockSpec((_BB, 1), lambda i: (i, 0))
    out = pl.pallas_call(
        _tc_body,
        grid=(B // _BB,),
        in_specs=[row_spec] * 8 + [col_spec] * 7 + [
            pl.BlockSpec((NREL, D), lambda i: (0, 0)),
            pl.BlockSpec((NREL, D * D), lambda i: (0, 0)),
            pl.BlockSpec((32, D), lambda i: (0, 0)),
            pl.BlockSpec((64, D), lambda i: (0, 0)),
            pl.BlockSpec((64, D), lambda i: (0, 0)),
        ],
        out_specs=pl.BlockSpec((1, 1), lambda i: (0, 0)),
        out_shape=jax.ShapeDtypeStruct((1, 1), jnp.float32),
        interpret=interpret,
    )(*gathered, *id_cols, r2, rel_W, M2d, *tails)
    return out[0, 0]


def _offsets(idx, n_rows):
    k = jnp.arange(B * D, dtype=jnp.int32)
    return (k % D) * n_rows + jnp.repeat(idx, D)


def kernel(user_W, item_W, entity_W, rel_W, trans_M,
           user_ids, item_pos_ids, item_neg_ids, h, r, pos_t, neg_t,
           is_train=1):
    i32 = lambda a: a.astype(jnp.int32)
    uid, ipid, inegid = i32(user_ids), i32(item_pos_ids), i32(item_neg_ids)
    hid, ptid, ntid = i32(h), i32(pos_t), i32(neg_t)

    user_flat, item_flat, entity_flat = _make_sc_detile()(
        user_W.T, item_W.T, entity_W.T)

    offs = [_offsets(uid, N_USR)] + [
        _offsets(a, N_BIG) for a in (ipid, inegid, hid, ptid, ntid)]

    flat_out = _make_sc_gather()(user_flat, item_flat, entity_flat, *offs)
    gathered = [f.reshape(B * D // 128, 128) for f in flat_out]

    def perm(a):
        return a.reshape(-1, _BB // 8, 8).transpose(0, 2, 1).reshape(B, 1)

    id_cols = [perm(a) for a in (uid, ipid, inegid, hid, ptid, ntid)]
    tails = [user_W[N_USR_VALID:], item_W[N_BIG_VALID:], entity_W[N_BIG_VALID:]]
    M2d = trans_M.reshape(NREL, D * D)
    r2 = perm(i32(r))
    return _tc_call(gathered, id_cols, r2, rel_W, M2d, tails)


# speedup vs baseline: 8.0240x; 1.1406x over previous
import functools

import jax
import jax.numpy as jnp
from jax import lax
from jax.experimental import pallas as pl
from jax.experimental.pallas import tpu as pltpu
from jax.experimental.pallas import tpu_sc as plsc

B = 16384
D = 16
N_BIG = 1000000
N_USR = 100000
NREL = 64
CF_LAMBDA = 1e-05
KG_LAMBDA = 1e-05

_NC, _NS = 2, 16
NW = _NC * _NS
BPW = B // NW
EPW = BPW * D


_CW = 16384


@functools.cache
def _make_sc_detile():
    mesh = plsc.VectorSubcoreMesh(core_axis_name="c", subcore_axis_name="s")

    SPAN_BIG = 999424 // NW
    SPAN_USR = 98304 // NW

    @functools.partial(
        pl.kernel,
        mesh=mesh,
        out_type=[
            jax.ShapeDtypeStruct((D * N_USR,), jnp.float32),
            jax.ShapeDtypeStruct((D * N_BIG,), jnp.float32),
            jax.ShapeDtypeStruct((D * N_BIG,), jnp.float32),
        ],
        scratch_types=[
            pltpu.VMEM((SPAN_BIG,), jnp.float32),
            pltpu.VMEM((SPAN_BIG,), jnp.float32),
            pltpu.VMEM((64,), jnp.float32),
            pltpu.SemaphoreType.DMA,
            pltpu.SemaphoreType.DMA,
            pltpu.SemaphoreType.DMA,
            pltpu.SemaphoreType.DMA,
        ],
    )
    def _sc_detile(userT, itemT, entityT, user_flat, item_flat, entity_flat,
                   vbuf0, vbuf1, zbuf, rsem0, rsem1, wsem0, wsem1):
        wid = lax.axis_index("s") * _NC + lax.axis_index("c")

        for i0 in range(0, 64, 16):
            zbuf[pl.ds(i0, 16)] = jnp.zeros((16,), jnp.float32)

        tasks = []
        for j in range(D):
            tasks.append((itemT, item_flat, N_BIG, j, SPAN_BIG))
            tasks.append((entityT, entity_flat, N_BIG, j, SPAN_BIG))
            tasks.append((userT, user_flat, N_USR, j, SPAN_USR))

        bufs = [vbuf0, vbuf1]
        rsems = [rsem0, rsem1]
        wsems = [wsem0, wsem1]
        pending = [None, None]
        for k, (src, dst, n, j, span) in enumerate(tasks):
            b = k % 2
            if pending[b] is not None:
                pending[b].wait()
            c0 = wid * span
            buf = bufs[b].at[pl.ds(0, span)]
            pltpu.async_copy(src.at[j, pl.ds(c0, span)], buf, rsems[b]).wait()
            pending[b] = pltpu.async_copy(
                buf, dst.at[pl.ds(j * n + c0, span)], wsems[b])
        for p in pending:
            p.wait()

        for j in range(D):
            @pl.when(wid == j)
            def _():
                pltpu.sync_copy(itemT.at[j, pl.ds(999424, 512)],
                                vbuf0.at[pl.ds(0, 512)])
                pltpu.sync_copy(vbuf0.at[pl.ds(0, 512)],
                                item_flat.at[pl.ds(j * N_BIG + 999424, 512)])
                pltpu.sync_copy(userT.at[j, pl.ds(98304, 1664)],
                                vbuf0.at[pl.ds(0, 1664)])
                pltpu.sync_copy(vbuf0.at[pl.ds(0, 1664)],
                                user_flat.at[pl.ds(j * N_USR + 98304, 1664)])
                pltpu.sync_copy(zbuf.at[pl.ds(0, 32)],
                                user_flat.at[pl.ds(j * N_USR + 99968, 32)])

            @pl.when(wid == D + j)
            def _():
                pltpu.sync_copy(entityT.at[j, pl.ds(999424, 512)],
                                vbuf0.at[pl.ds(0, 512)])
                pltpu.sync_copy(vbuf0.at[pl.ds(0, 512)],
                                entity_flat.at[pl.ds(j * N_BIG + 999424, 512)])
                pltpu.sync_copy(zbuf,
                                item_flat.at[pl.ds(j * N_BIG + 999936, 64)])
                pltpu.sync_copy(zbuf,
                                entity_flat.at[pl.ds(j * N_BIG + 999936, 64)])

    return _sc_detile


@functools.cache
def _make_sc_gather():
    mesh = plsc.VectorSubcoreMesh(core_axis_name="c", subcore_axis_name="s")

    @functools.partial(
        pl.kernel,
        mesh=mesh,
        out_type=[jax.ShapeDtypeStruct((B * D,), jnp.float32)] * 8,
        scratch_types=[
            pltpu.VMEM((EPW,), jnp.int32),
            pltpu.VMEM((EPW,), jnp.float32),
            pltpu.SemaphoreType.DMA,
        ],
    )
    def _sc_gather(user_flat, item_flat, entity_flat,
                   off_u, off_ip, off_ineg, off_h, off_pt, off_nt,
                   u_out, ip_out, ineg_out, ipk_out, inegk_out,
                   he_out, pt_out, nt_out,
                   idx_v, rows_v, sem):
        wid = lax.axis_index("s") * _NC + lax.axis_index("c")
        base = wid * EPW

        def load_off(off):
            pltpu.sync_copy(off.at[pl.ds(base, EPW)], idx_v)

        def gather_to(tab, out):
            pltpu.async_copy(tab.at[idx_v], rows_v, sem).wait()
            pltpu.sync_copy(rows_v, out.at[pl.ds(base, EPW)])

        load_off(off_u)
        gather_to(user_flat, u_out)
        load_off(off_ip)
        gather_to(item_flat, ip_out)
        gather_to(entity_flat, ipk_out)
        load_off(off_ineg)
        gather_to(item_flat, ineg_out)
        gather_to(entity_flat, inegk_out)
        load_off(off_h)
        gather_to(entity_flat, he_out)
        load_off(off_pt)
        gather_to(entity_flat, pt_out)
        load_off(off_nt)
        gather_to(entity_flat, nt_out)

    return _sc_gather


N_USR_VALID = 99968
N_BIG_VALID = 999936


def _patch(x, id_col, n_valid, tail_tab, width):
    oh = (id_col - n_valid == lax.broadcasted_iota(jnp.int32, (1, width), 1))
    patched = jnp.dot(oh.astype(jnp.float32), tail_tab,
                      preferred_element_type=jnp.float32)
    return jnp.where(id_col >= n_valid, patched, x)


def _tc_body(u_ref, ip_ref, ineg_ref, ipk_ref, inegk_ref,
             he_ref, pt_ref, nt_ref,
             uid_ref, ipid_ref, inegid_ref, hid_ref, ptid_ref, ntid_ref,
             r_ref, relW_ref, M2d_ref,
             tailu_ref, taili_ref, taile_ref, out_ref):
    i = pl.program_id(0)

    @pl.when(i == 0)
    def _init():
        out_ref[...] = jnp.zeros((1, 1), jnp.float32)

    tailu = tailu_ref[...]
    taili = taili_ref[...]
    taile = taile_ref[...]
    uid = uid_ref[...]
    ipid = ipid_ref[...]
    inegid = inegid_ref[...]

    def unpack(ref):
        x = ref[...]
        return jnp.concatenate(
            [x[:, D * s:D * (s + 1)] for s in range(8)], axis=0)

    u = _patch(unpack(u_ref), uid, N_USR_VALID, tailu, 32)
    ip = _patch(unpack(ip_ref), ipid, N_BIG_VALID, taili, 64)
    ipk = _patch(unpack(ipk_ref), ipid, N_BIG_VALID, taile, 64)
    ineg = _patch(unpack(ineg_ref), inegid, N_BIG_VALID, taili, 64)
    inegk = _patch(unpack(inegk_ref), inegid, N_BIG_VALID, taile, 64)
    pos_cf = ip + ipk
    neg_cf = ineg + inegk
    pos_s = jnp.sum(u * pos_cf, axis=1, keepdims=True)
    neg_s = jnp.sum(u * neg_cf, axis=1, keepdims=True)
    x = pos_s - neg_s
    sig = 1.0 / (1.0 + jnp.exp(-x))
    cf_term = -jnp.log(1e-10 + sig)
    l2_cf = 0.5 * (jnp.sum(u * u) + jnp.sum(pos_cf * pos_cf)
                   + jnp.sum(neg_cf * neg_cf))

    r_col = r_ref[...]
    k_row = lax.broadcasted_iota(jnp.int32, (1, NREL), 1)
    onehot = (r_col == k_row).astype(jnp.float32)
    Weff = jnp.dot(onehot, M2d_ref[...],
                   preferred_element_type=jnp.float32)
    re = jnp.dot(onehot, relW_ref[...],
                 preferred_element_type=jnp.float32)

    c1 = lax.broadcasted_iota(jnp.int32, (D, D * D), 1)
    d1 = lax.broadcasted_iota(jnp.int32, (D, D * D), 0)
    R = ((c1 // D) == d1).astype(jnp.float32)
    c2 = lax.broadcasted_iota(jnp.int32, (D * D, D), 0)
    j2 = lax.broadcasted_iota(jnp.int32, (D * D, D), 1)
    S = ((c2 % D) == j2).astype(jnp.float32)

    def proj(x16):
        xr = jnp.dot(x16, R, preferred_element_type=jnp.float32)
        return jnp.dot(xr * Weff, S, preferred_element_type=jnp.float32)

    rh = proj(_patch(unpack(he_ref), hid_ref[...], N_BIG_VALID, taile, 64))
    rpt = proj(_patch(unpack(pt_ref), ptid_ref[...], N_BIG_VALID, taile, 64))
    rnt = proj(_patch(unpack(nt_ref), ntid_ref[...], N_BIG_VALID, taile, 64))

    def normz(v):
        n = jnp.sqrt(jnp.sum(v * v, axis=1, keepdims=True))
        return v / jnp.maximum(n, 1e-12)

    re_n = normz(re)
    rh_n = normz(rh)
    rpt_n = normz(rpt)
    rnt_n = normz(rnt)

    dpos = rh_n + re_n - rpt_n
    dneg = rh_n + re_n - rnt_n
    pos_sc = jnp.sqrt(jnp.sum(dpos * dpos, axis=1, keepdims=True))
    neg_sc = jnp.sqrt(jnp.sum(dneg * dneg, axis=1, keepdims=True))
    kg_term = jnp.maximum(pos_sc - neg_sc + 1.0, 0.0)
    l2_kg = 0.5 * (jnp.sum(rh_n * rh_n) + jnp.sum(re_n * re_n)
                   + jnp.sum(rpt_n * rpt_n) + jnp.sum(rnt_n * rnt_n))

    block_total = (jnp.sum(cf_term) + CF_LAMBDA * l2_cf
                   + jnp.sum(kg_term) + KG_LAMBDA * l2_kg)
    out_ref[...] += jnp.reshape(block_total * (1.0 / B), (1, 1))


_BB = 2048


def _tc_call(gathered, id_cols, r2, rel_W, M2d, tails, interpret=False):
    row_spec = pl.BlockSpec((_BB // 8, 128), lambda i: (i, 0))
    col_spec = pl.BlockSpec((_BB, 1), lambda i: (i, 0))
    out = pl.pallas_call(
        _tc_body,
        grid=(B // _BB,),
        in_specs=[row_spec] * 8 + [col_spec] * 7 + [
            pl.BlockSpec((NREL, D), lambda i: (0, 0)),
            pl.BlockSpec((NREL, D * D), lambda i: (0, 0)),
            pl.BlockSpec((32, D), lambda i: (0, 0)),
            pl.BlockSpec((64, D), lambda i: (0, 0)),
            pl.BlockSpec((64, D), lambda i: (0, 0)),
        ],
        out_specs=pl.BlockSpec((1, 1), lambda i: (0, 0)),
        out_shape=jax.ShapeDtypeStruct((1, 1), jnp.float32),
        interpret=interpret,
    )(*gathered, *id_cols, r2, rel_W, M2d, *tails)
    return out[0, 0]


def _offsets(idx, n_rows):
    k = jnp.arange(B * D, dtype=jnp.int32)
    return (k % D) * n_rows + jnp.repeat(idx, D)


def kernel(user_W, item_W, entity_W, rel_W, trans_M,
           user_ids, item_pos_ids, item_neg_ids, h, r, pos_t, neg_t,
           is_train=1):
    i32 = lambda a: a.astype(jnp.int32)
    uid, ipid, inegid = i32(user_ids), i32(item_pos_ids), i32(item_neg_ids)
    hid, ptid, ntid = i32(h), i32(pos_t), i32(neg_t)

    user_flat, item_flat, entity_flat = _make_sc_detile()(
        user_W.T, item_W.T, entity_W.T)

    offs = [_offsets(uid, N_USR)] + [
        _offsets(a, N_BIG) for a in (ipid, inegid, hid, ptid, ntid)]

    flat_out = _make_sc_gather()(user_flat, item_flat, entity_flat, *offs)
    gathered = [f.reshape(B * D // 128, 128) for f in flat_out]

    def perm(a):
        return a.reshape(-1, _BB // 8, 8).transpose(0, 2, 1).reshape(B, 1)

    id_cols = [perm(a) for a in (uid, ipid, inegid, hid, ptid, ntid)]
    tails = [user_W[N_USR_VALID:], item_W[N_BIG_VALID:], entity_W[N_BIG_VALID:]]
    M2d = trans_M.reshape(NREL, D * D)
    r2 = perm(i32(r))
    return _tc_call(gathered, id_cols, r2, rel_W, M2d, tails)


# speedup vs baseline: 8.1601x; 1.0170x over previous
import functools

import jax
import jax.numpy as jnp
from jax import lax
from jax.experimental import pallas as pl
from jax.experimental.pallas import tpu as pltpu
from jax.experimental.pallas import tpu_sc as plsc

B = 16384
D = 16
N_BIG = 1000000
N_USR = 100000
NREL = 64
CF_LAMBDA = 1e-05
KG_LAMBDA = 1e-05

_NC, _NS = 2, 16
NW = _NC * _NS
BPW = B // NW
EPW = BPW * D


_CW = 16384


@functools.cache
def _make_sc_detile():
    mesh = plsc.VectorSubcoreMesh(core_axis_name="c", subcore_axis_name="s")

    SPAN_BIG = 999424 // NW
    SPAN_USR = 98304 // NW

    @functools.partial(
        pl.kernel,
        mesh=mesh,
        out_type=[
            jax.ShapeDtypeStruct((D * N_USR,), jnp.float32),
            jax.ShapeDtypeStruct((D * N_BIG,), jnp.float32),
            jax.ShapeDtypeStruct((D * N_BIG,), jnp.float32),
        ],
        scratch_types=[
            pltpu.VMEM((SPAN_BIG,), jnp.float32),
            pltpu.VMEM((SPAN_BIG,), jnp.float32),
            pltpu.VMEM((64,), jnp.float32),
            pltpu.SemaphoreType.DMA,
            pltpu.SemaphoreType.DMA,
            pltpu.SemaphoreType.DMA,
            pltpu.SemaphoreType.DMA,
        ],
    )
    def _sc_detile(userT, itemT, entityT, user_flat, item_flat, entity_flat,
                   vbuf0, vbuf1, zbuf, rsem0, rsem1, wsem0, wsem1):
        wid = lax.axis_index("s") * _NC + lax.axis_index("c")

        for i0 in range(0, 64, 16):
            zbuf[pl.ds(i0, 16)] = jnp.zeros((16,), jnp.float32)

        tasks = []
        for j in range(D):
            tasks.append((itemT, item_flat, N_BIG, j, SPAN_BIG))
            tasks.append((entityT, entity_flat, N_BIG, j, SPAN_BIG))
            tasks.append((userT, user_flat, N_USR, j, SPAN_USR))

        bufs = [vbuf0, vbuf1]
        rsems = [rsem0, rsem1]
        wsems = [wsem0, wsem1]
        pending = [None, None]
        for k, (src, dst, n, j, span) in enumerate(tasks):
            b = k % 2
            if pending[b] is not None:
                pending[b].wait()
            c0 = wid * span
            buf = bufs[b].at[pl.ds(0, span)]
            pltpu.async_copy(src.at[j, pl.ds(c0, span)], buf, rsems[b]).wait()
            pending[b] = pltpu.async_copy(
                buf, dst.at[pl.ds(j * n + c0, span)], wsems[b])
        for p in pending:
            if p is not None:
                p.wait()

        for j in range(D):
            @pl.when(wid == j)
            def _():
                pltpu.sync_copy(itemT.at[j, pl.ds(999424, 512)],
                                vbuf0.at[pl.ds(0, 512)])
                pltpu.sync_copy(vbuf0.at[pl.ds(0, 512)],
                                item_flat.at[pl.ds(j * N_BIG + 999424, 512)])
                pltpu.sync_copy(userT.at[j, pl.ds(98304, 1664)],
                                vbuf0.at[pl.ds(0, 1664)])
                pltpu.sync_copy(vbuf0.at[pl.ds(0, 1664)],
                                user_flat.at[pl.ds(j * N_USR + 98304, 1664)])
                pltpu.sync_copy(zbuf.at[pl.ds(0, 32)],
                                user_flat.at[pl.ds(j * N_USR + 99968, 32)])

            @pl.when(wid == D + j)
            def _():
                pltpu.sync_copy(entityT.at[j, pl.ds(999424, 512)],
                                vbuf0.at[pl.ds(0, 512)])
                pltpu.sync_copy(vbuf0.at[pl.ds(0, 512)],
                                entity_flat.at[pl.ds(j * N_BIG + 999424, 512)])
                pltpu.sync_copy(zbuf,
                                item_flat.at[pl.ds(j * N_BIG + 999936, 64)])
                pltpu.sync_copy(zbuf,
                                entity_flat.at[pl.ds(j * N_BIG + 999936, 64)])

    return _sc_detile


@functools.cache
def _make_sc_gather():
    mesh = plsc.VectorSubcoreMesh(core_axis_name="c", subcore_axis_name="s")

    @functools.partial(
        pl.kernel,
        mesh=mesh,
        out_type=[jax.ShapeDtypeStruct((B * D,), jnp.float32)] * 8,
        scratch_types=[
            pltpu.VMEM((EPW,), jnp.int32),
            pltpu.VMEM((EPW,), jnp.int32),
            pltpu.VMEM((EPW,), jnp.float32),
            pltpu.VMEM((EPW,), jnp.float32),
            pltpu.SemaphoreType.DMA,
            pltpu.SemaphoreType.DMA,
            pltpu.SemaphoreType.DMA,
            pltpu.SemaphoreType.DMA,
            pltpu.SemaphoreType.DMA,
        ],
    )
    def _sc_gather(user_flat, item_flat, entity_flat,
                   off_u, off_ip, off_ineg, off_h, off_pt, off_nt,
                   u_out, ip_out, ineg_out, ipk_out, inegk_out,
                   he_out, pt_out, nt_out,
                   idx0, idx1, rows0, rows1, isem0, isem1, gsem,
                   wsem0, wsem1):
        wid = lax.axis_index("s") * _NC + lax.axis_index("c")
        base = wid * EPW

        groups = [
            (off_u, [(user_flat, u_out)]),
            (off_ip, [(item_flat, ip_out), (entity_flat, ipk_out)]),
            (off_ineg, [(item_flat, ineg_out), (entity_flat, inegk_out)]),
            (off_h, [(entity_flat, he_out)]),
            (off_pt, [(entity_flat, pt_out)]),
            (off_nt, [(entity_flat, nt_out)]),
        ]
        idxs = [idx0, idx1]
        isems = [isem0, isem1]
        rows = [rows0, rows1]
        wsems = [wsem0, wsem1]

        ipend = [None, None]
        ipend[0] = pltpu.async_copy(
            groups[0][0].at[pl.ds(base, EPW)], idxs[0], isems[0])
        wpend = [None, None]
        k = 0
        for g, (off, pairs) in enumerate(groups):
            gb = g % 2
            ipend[gb].wait()
            if g + 1 < len(groups):
                nb = (g + 1) % 2
                ipend[nb] = pltpu.async_copy(
                    groups[g + 1][0].at[pl.ds(base, EPW)], idxs[nb],
                    isems[nb])
            for tab, out in pairs:
                rb = k % 2
                if wpend[rb] is not None:
                    wpend[rb].wait()
                pltpu.async_copy(tab.at[idxs[gb]], rows[rb], gsem).wait()
                wpend[rb] = pltpu.async_copy(
                    rows[rb], out.at[pl.ds(base, EPW)], wsems[rb])
                k += 1
        for p in wpend:
            if p is not None:
                p.wait()

    return _sc_gather


N_USR_VALID = 99968
N_BIG_VALID = 999936


def _patch(x, id_col, n_valid, tail_tab, width):
    oh = (id_col - n_valid == lax.broadcasted_iota(jnp.int32, (1, width), 1))
    patched = jnp.dot(oh.astype(jnp.float32), tail_tab,
                      preferred_element_type=jnp.float32)
    return jnp.where(id_col >= n_valid, patched, x)


def _tc_body(u_ref, ip_ref, ineg_ref, ipk_ref, inegk_ref,
             he_ref, pt_ref, nt_ref,
             uid_ref, ipid_ref, inegid_ref, hid_ref, ptid_ref, ntid_ref,
             r_ref, relW_ref, M2d_ref,
             tailu_ref, taili_ref, taile_ref, out_ref):
    i = pl.program_id(0)

    @pl.when(i == 0)
    def _init():
        out_ref[...] = jnp.zeros((1, 1), jnp.float32)

    tailu = tailu_ref[...]
    taili = taili_ref[...]
    taile = taile_ref[...]
    uid = uid_ref[...]
    ipid = ipid_ref[...]
    inegid = inegid_ref[...]

    def unpack(ref):
        x = ref[...]
        return jnp.concatenate(
            [x[:, D * s:D * (s + 1)] for s in range(8)], axis=0)

    u = _patch(unpack(u_ref), uid, N_USR_VALID, tailu, 32)
    ip = _patch(unpack(ip_ref), ipid, N_BIG_VALID, taili, 64)
    ipk = _patch(unpack(ipk_ref), ipid, N_BIG_VALID, taile, 64)
    ineg = _patch(unpack(ineg_ref), inegid, N_BIG_VALID, taili, 64)
    inegk = _patch(unpack(inegk_ref), inegid, N_BIG_VALID, taile, 64)
    pos_cf = ip + ipk
    neg_cf = ineg + inegk
    pos_s = jnp.sum(u * pos_cf, axis=1, keepdims=True)
    neg_s = jnp.sum(u * neg_cf, axis=1, keepdims=True)
    x = pos_s - neg_s
    sig = 1.0 / (1.0 + jnp.exp(-x))
    cf_term = -jnp.log(1e-10 + sig)
    l2_cf = 0.5 * (jnp.sum(u * u) + jnp.sum(pos_cf * pos_cf)
                   + jnp.sum(neg_cf * neg_cf))

    r_col = r_ref[...]
    k_row = lax.broadcasted_iota(jnp.int32, (1, NREL), 1)
    onehot = (r_col == k_row).astype(jnp.float32)
    Weff = jnp.dot(onehot, M2d_ref[...],
                   preferred_element_type=jnp.float32)
    re = jnp.dot(onehot, relW_ref[...],
                 preferred_element_type=jnp.float32)

    c1 = lax.broadcasted_iota(jnp.int32, (D, D * D), 1)
    d1 = lax.broadcasted_iota(jnp.int32, (D, D * D), 0)
    R = ((c1 // D) == d1).astype(jnp.float32)
    c2 = lax.broadcasted_iota(jnp.int32, (D * D, D), 0)
    j2 = lax.broadcasted_iota(jnp.int32, (D * D, D), 1)
    S = ((c2 % D) == j2).astype(jnp.float32)

    def proj(x16):
        xr = jnp.dot(x16, R, preferred_element_type=jnp.float32)
        return jnp.dot(xr * Weff, S, preferred_element_type=jnp.float32)

    rh = proj(_patch(unpack(he_ref), hid_ref[...], N_BIG_VALID, taile, 64))
    rpt = proj(_patch(unpack(pt_ref), ptid_ref[...], N_BIG_VALID, taile, 64))
    rnt = proj(_patch(unpack(nt_ref), ntid_ref[...], N_BIG_VALID, taile, 64))

    def normz(v):
        n = jnp.sqrt(jnp.sum(v * v, axis=1, keepdims=True))
        return v / jnp.maximum(n, 1e-12)

    re_n = normz(re)
    rh_n = normz(rh)
    rpt_n = normz(rpt)
    rnt_n = normz(rnt)

    dpos = rh_n + re_n - rpt_n
    dneg = rh_n + re_n - rnt_n
    pos_sc = jnp.sqrt(jnp.sum(dpos * dpos, axis=1, keepdims=True))
    neg_sc = jnp.sqrt(jnp.sum(dneg * dneg, axis=1, keepdims=True))
    kg_term = jnp.maximum(pos_sc - neg_sc + 1.0, 0.0)
    l2_kg = 0.5 * (jnp.sum(rh_n * rh_n) + jnp.sum(re_n * re_n)
                   + jnp.sum(rpt_n * rpt_n) + jnp.sum(rnt_n * rnt_n))

    block_total = (jnp.sum(cf_term) + CF_LAMBDA * l2_cf
                   + jnp.sum(kg_term) + KG_LAMBDA * l2_kg)
    out_ref[...] += jnp.reshape(block_total * (1.0 / B), (1, 1))


_BB = 2048


def _tc_call(gathered, id_cols, r2, rel_W, M2d, tails, interpret=False):
    row_spec = pl.BlockSpec((_BB // 8, 128), lambda i: (i, 0))
    col_spec = pl.BlockSpec((_BB, 1), lambda i: (i, 0))
    out = pl.pallas_call(
        _tc_body,
        grid=(B // _BB,),
        in_specs=[row_spec] * 8 + [col_spec] * 7 + [
            pl.BlockSpec((NREL, D), lambda i: (0, 0)),
            pl.BlockSpec((NREL, D * D), lambda i: (0, 0)),
            pl.BlockSpec((32, D), lambda i: (0, 0)),
            pl.BlockSpec((64, D), lambda i: (0, 0)),
            pl.BlockSpec((64, D), lambda i: (0, 0)),
        ],
        out_specs=pl.BlockSpec((1, 1), lambda i: (0, 0)),
        out_shape=jax.ShapeDtypeStruct((1, 1), jnp.float32),
        interpret=interpret,
    )(*gathered, *id_cols, r2, rel_W, M2d, *tails)
    return out[0, 0]


def _offsets(idx, n_rows):
    k = jnp.arange(B * D, dtype=jnp.int32)
    return (k % D) * n_rows + jnp.repeat(idx, D)


def kernel(user_W, item_W, entity_W, rel_W, trans_M,
           user_ids, item_pos_ids, item_neg_ids, h, r, pos_t, neg_t,
           is_train=1):
    i32 = lambda a: a.astype(jnp.int32)
    uid, ipid, inegid = i32(user_ids), i32(item_pos_ids), i32(item_neg_ids)
    hid, ptid, ntid = i32(h), i32(pos_t), i32(neg_t)

    user_flat, item_flat, entity_flat = _make_sc_detile()(
        user_W.T, item_W.T, entity_W.T)

    offs = [_offsets(uid, N_USR)] + [
        _offsets(a, N_BIG) for a in (ipid, inegid, hid, ptid, ntid)]

    flat_out = _make_sc_gather()(user_flat, item_flat, entity_flat, *offs)
    gathered = [f.reshape(B * D // 128, 128) for f in flat_out]

    def perm(a):
        return a.reshape(-1, _BB // 8, 8).transpose(0, 2, 1).reshape(B, 1)

    id_cols = [perm(a) for a in (uid, ipid, inegid, hid, ptid, ntid)]
    tails = [user_W[N_USR_VALID:], item_W[N_BIG_VALID:], entity_W[N_BIG_VALID:]]
    M2d = trans_M.reshape(NREL, D * D)
    r2 = perm(i32(r))
    return _tc_call(gathered, id_cols, r2, rel_W, M2d, tails)


# speedup vs baseline: 8.5346x; 1.0459x over previous
import functools

import jax
import jax.numpy as jnp
from jax import lax
from jax.experimental import pallas as pl
from jax.experimental.pallas import tpu as pltpu
from jax.experimental.pallas import tpu_sc as plsc

B = 16384
D = 16
N_BIG = 1000000
N_USR = 100000
NREL = 64
CF_LAMBDA = 1e-05
KG_LAMBDA = 1e-05

_NC, _NS = 2, 16
NW = _NC * _NS
BPW = B // NW
EPW = BPW * D


_CW = 16384


@functools.cache
def _make_sc_detile():
    mesh = plsc.VectorSubcoreMesh(core_axis_name="c", subcore_axis_name="s")

    SPAN_BIG = 999424 // NW
    SPAN_USR = 98304 // NW

    @functools.partial(
        pl.kernel,
        mesh=mesh,
        out_type=[
            jax.ShapeDtypeStruct((D * N_USR,), jnp.float32),
            jax.ShapeDtypeStruct((D * N_BIG,), jnp.float32),
            jax.ShapeDtypeStruct((D * N_BIG,), jnp.float32),
        ],
        scratch_types=[
            pltpu.VMEM((SPAN_BIG,), jnp.float32),
            pltpu.VMEM((SPAN_BIG,), jnp.float32),
            pltpu.VMEM((64,), jnp.float32),
            pltpu.SemaphoreType.DMA,
            pltpu.SemaphoreType.DMA,
            pltpu.SemaphoreType.DMA,
            pltpu.SemaphoreType.DMA,
        ],
    )
    def _sc_detile(userT, itemT, entityT, user_flat, item_flat, entity_flat,
                   vbuf0, vbuf1, zbuf, rsem0, rsem1, wsem0, wsem1):
        wid = lax.axis_index("s") * _NC + lax.axis_index("c")

        for i0 in range(0, 64, 16):
            zbuf[pl.ds(i0, 16)] = jnp.zeros((16,), jnp.float32)

        tasks = []
        for j in range(D):
            tasks.append((itemT, item_flat, N_BIG, j, SPAN_BIG))
            tasks.append((entityT, entity_flat, N_BIG, j, SPAN_BIG))
            tasks.append((userT, user_flat, N_USR, j, SPAN_USR))

        bufs = [vbuf0, vbuf1]
        rsems = [rsem0, rsem1]
        wsems = [wsem0, wsem1]
        pending = [None, None]
        for k, (src, dst, n, j, span) in enumerate(tasks):
            b = k % 2
            if pending[b] is not None:
                pending[b].wait()
            c0 = wid * span
            buf = bufs[b].at[pl.ds(0, span)]
            pltpu.async_copy(src.at[j, pl.ds(c0, span)], buf, rsems[b]).wait()
            pending[b] = pltpu.async_copy(
                buf, dst.at[pl.ds(j * n + c0, span)], wsems[b])
        for p in pending:
            if p is not None:
                p.wait()

        for j in range(D):
            @pl.when(wid == j)
            def _():
                pltpu.sync_copy(itemT.at[j, pl.ds(999424, 512)],
                                vbuf0.at[pl.ds(0, 512)])
                pltpu.sync_copy(vbuf0.at[pl.ds(0, 512)],
                                item_flat.at[pl.ds(j * N_BIG + 999424, 512)])
                pltpu.sync_copy(userT.at[j, pl.ds(98304, 1664)],
                                vbuf0.at[pl.ds(0, 1664)])
                pltpu.sync_copy(vbuf0.at[pl.ds(0, 1664)],
                                user_flat.at[pl.ds(j * N_USR + 98304, 1664)])
                pltpu.sync_copy(zbuf.at[pl.ds(0, 32)],
                                user_flat.at[pl.ds(j * N_USR + 99968, 32)])

            @pl.when(wid == D + j)
            def _():
                pltpu.sync_copy(entityT.at[j, pl.ds(999424, 512)],
                                vbuf0.at[pl.ds(0, 512)])
                pltpu.sync_copy(vbuf0.at[pl.ds(0, 512)],
                                entity_flat.at[pl.ds(j * N_BIG + 999424, 512)])
                pltpu.sync_copy(zbuf,
                                item_flat.at[pl.ds(j * N_BIG + 999936, 64)])
                pltpu.sync_copy(zbuf,
                                entity_flat.at[pl.ds(j * N_BIG + 999936, 64)])

    return _sc_detile


@functools.cache
def _make_sc_gather():
    mesh = plsc.VectorSubcoreMesh(core_axis_name="c", subcore_axis_name="s")

    @functools.partial(
        pl.kernel,
        mesh=mesh,
        out_type=[jax.ShapeDtypeStruct((B * D,), jnp.float32)] * 8,
        scratch_types=[
            pltpu.VMEM((EPW,), jnp.int32),
            pltpu.VMEM((EPW,), jnp.int32),
            pltpu.VMEM((EPW,), jnp.float32),
            pltpu.VMEM((EPW,), jnp.float32),
            pltpu.SemaphoreType.DMA,
            pltpu.SemaphoreType.DMA,
            pltpu.SemaphoreType.DMA,
            pltpu.SemaphoreType.DMA,
            pltpu.SemaphoreType.DMA,
        ],
    )
    def _sc_gather(user_flat, item_flat, entity_flat,
                   off_u, off_ip, off_ineg, off_h, off_pt, off_nt,
                   u_out, ip_out, ineg_out, ipk_out, inegk_out,
                   he_out, pt_out, nt_out,
                   idx0, idx1, rows0, rows1, isem0, isem1, gsem,
                   wsem0, wsem1):
        wid = lax.axis_index("s") * _NC + lax.axis_index("c")
        base = wid * EPW

        groups = [
            (off_u, [(user_flat, u_out)]),
            (off_ip, [(item_flat, ip_out), (entity_flat, ipk_out)]),
            (off_ineg, [(item_flat, ineg_out), (entity_flat, inegk_out)]),
            (off_h, [(entity_flat, he_out)]),
            (off_pt, [(entity_flat, pt_out)]),
            (off_nt, [(entity_flat, nt_out)]),
        ]
        idxs = [idx0, idx1]
        isems = [isem0, isem1]
        rows = [rows0, rows1]
        wsems = [wsem0, wsem1]

        ipend = [None, None]
        ipend[0] = pltpu.async_copy(
            groups[0][0].at[pl.ds(base, EPW)], idxs[0], isems[0])
        wpend = [None, None]
        k = 0
        for g, (off, pairs) in enumerate(groups):
            gb = g % 2
            ipend[gb].wait()
            if g + 1 < len(groups):
                nb = (g + 1) % 2
                ipend[nb] = pltpu.async_copy(
                    groups[g + 1][0].at[pl.ds(base, EPW)], idxs[nb],
                    isems[nb])
            for tab, out in pairs:
                rb = k % 2
                if wpend[rb] is not None:
                    wpend[rb].wait()
                pltpu.async_copy(tab.at[idxs[gb]], rows[rb], gsem).wait()
                wpend[rb] = pltpu.async_copy(
                    rows[rb], out.at[pl.ds(base, EPW)], wsems[rb])
                k += 1
        for p in wpend:
            if p is not None:
                p.wait()

    return _sc_gather


N_USR_VALID = 99968
N_BIG_VALID = 999936


def _patch(x, id_col, n_valid, tail_tab, width):
    oh = (id_col - n_valid == lax.broadcasted_iota(jnp.int32, (1, width), 1))
    patched = jnp.dot(oh.astype(jnp.float32), tail_tab,
                      preferred_element_type=jnp.float32)
    return jnp.where(id_col >= n_valid, patched, x)


def _tc_body(u_ref, ip_ref, ineg_ref, ipk_ref, inegk_ref,
             he_ref, pt_ref, nt_ref,
             ids_ref, relW_ref, M2d_ref,
             tailu_ref, taili_ref, taile_ref, out_ref):
    i = pl.program_id(0)

    @pl.when(i == 0)
    def _init():
        out_ref[...] = jnp.zeros((1, 1), jnp.float32)

    tailu = tailu_ref[...]
    taili = taili_ref[...]
    taile = taile_ref[...]
    ids = ids_ref[...]
    uid = ids[:, 0:1]
    ipid = ids[:, 1:2]
    inegid = ids[:, 2:3]
    hid = ids[:, 3:4]
    ptid = ids[:, 4:5]
    ntid = ids[:, 5:6]
    r_col = ids[:, 6:7]

    def unpack(ref):
        x = ref[...]
        return jnp.concatenate(
            [x[:, D * s:D * (s + 1)] for s in range(8)], axis=0)

    u = _patch(unpack(u_ref), uid, N_USR_VALID, tailu, 32)
    ip = _patch(unpack(ip_ref), ipid, N_BIG_VALID, taili, 64)
    ipk = _patch(unpack(ipk_ref), ipid, N_BIG_VALID, taile, 64)
    ineg = _patch(unpack(ineg_ref), inegid, N_BIG_VALID, taili, 64)
    inegk = _patch(unpack(inegk_ref), inegid, N_BIG_VALID, taile, 64)
    pos_cf = ip + ipk
    neg_cf = ineg + inegk
    pos_s = jnp.sum(u * pos_cf, axis=1, keepdims=True)
    neg_s = jnp.sum(u * neg_cf, axis=1, keepdims=True)
    x = pos_s - neg_s
    sig = 1.0 / (1.0 + jnp.exp(-x))
    cf_term = -jnp.log(1e-10 + sig)
    l2_cf = 0.5 * (jnp.sum(u * u) + jnp.sum(pos_cf * pos_cf)
                   + jnp.sum(neg_cf * neg_cf))

    k_row = lax.broadcasted_iota(jnp.int32, (1, NREL), 1)
    onehot = (r_col == k_row).astype(jnp.float32)
    Weff = jnp.dot(onehot, M2d_ref[...],
                   preferred_element_type=jnp.float32)
    re = jnp.dot(onehot, relW_ref[...],
                 preferred_element_type=jnp.float32)

    c1 = lax.broadcasted_iota(jnp.int32, (D, D * D), 1)
    d1 = lax.broadcasted_iota(jnp.int32, (D, D * D), 0)
    R = ((c1 // D) == d1).astype(jnp.float32)
    c2 = lax.broadcasted_iota(jnp.int32, (D * D, D), 0)
    j2 = lax.broadcasted_iota(jnp.int32, (D * D, D), 1)
    S = ((c2 % D) == j2).astype(jnp.float32)

    def proj(x16):
        xr = jnp.dot(x16, R, preferred_element_type=jnp.float32)
        return jnp.dot(xr * Weff, S, preferred_element_type=jnp.float32)

    rh = proj(_patch(unpack(he_ref), hid, N_BIG_VALID, taile, 64))
    rpt = proj(_patch(unpack(pt_ref), ptid, N_BIG_VALID, taile, 64))
    rnt = proj(_patch(unpack(nt_ref), ntid, N_BIG_VALID, taile, 64))

    def normz(v):
        n = jnp.sqrt(jnp.sum(v * v, axis=1, keepdims=True))
        return v / jnp.maximum(n, 1e-12)

    re_n = normz(re)
    rh_n = normz(rh)
    rpt_n = normz(rpt)
    rnt_n = normz(rnt)

    dpos = rh_n + re_n - rpt_n
    dneg = rh_n + re_n - rnt_n
    pos_sc = jnp.sqrt(jnp.sum(dpos * dpos, axis=1, keepdims=True))
    neg_sc = jnp.sqrt(jnp.sum(dneg * dneg, axis=1, keepdims=True))
    kg_term = jnp.maximum(pos_sc - neg_sc + 1.0, 0.0)
    l2_kg = 0.5 * (jnp.sum(rh_n * rh_n) + jnp.sum(re_n * re_n)
                   + jnp.sum(rpt_n * rpt_n) + jnp.sum(rnt_n * rnt_n))

    block_total = (jnp.sum(cf_term) + CF_LAMBDA * l2_cf
                   + jnp.sum(kg_term) + KG_LAMBDA * l2_kg)
    out_ref[...] += jnp.reshape(block_total * (1.0 / B), (1, 1))


_BB = 2048


def _tc_call(gathered, ids8, rel_W, M2d, tails, interpret=False):
    row_spec = pl.BlockSpec((_BB // 8, 128), lambda i: (i, 0))
    out = pl.pallas_call(
        _tc_body,
        grid=(B // _BB,),
        in_specs=[row_spec] * 8 + [
            pl.BlockSpec((_BB, 8), lambda i: (i, 0)),
        ] + [
            pl.BlockSpec((NREL, D), lambda i: (0, 0)),
            pl.BlockSpec((NREL, D * D), lambda i: (0, 0)),
            pl.BlockSpec((32, D), lambda i: (0, 0)),
            pl.BlockSpec((64, D), lambda i: (0, 0)),
            pl.BlockSpec((64, D), lambda i: (0, 0)),
        ],
        out_specs=pl.BlockSpec((1, 1), lambda i: (0, 0)),
        out_shape=jax.ShapeDtypeStruct((1, 1), jnp.float32),
        interpret=interpret,
    )(*gathered, ids8, rel_W, M2d, *tails)
    return out[0, 0]


def _offsets(idx, n_rows):
    k = jnp.arange(B * D, dtype=jnp.int32)
    return (k % D) * n_rows + jnp.repeat(idx, D)


def kernel(user_W, item_W, entity_W, rel_W, trans_M,
           user_ids, item_pos_ids, item_neg_ids, h, r, pos_t, neg_t,
           is_train=1):
    i32 = lambda a: a.astype(jnp.int32)
    uid, ipid, inegid = i32(user_ids), i32(item_pos_ids), i32(item_neg_ids)
    hid, ptid, ntid = i32(h), i32(pos_t), i32(neg_t)

    user_flat, item_flat, entity_flat = _make_sc_detile()(
        user_W.T, item_W.T, entity_W.T)

    offs = [_offsets(uid, N_USR)] + [
        _offsets(a, N_BIG) for a in (ipid, inegid, hid, ptid, ntid)]

    flat_out = _make_sc_gather()(user_flat, item_flat, entity_flat, *offs)
    gathered = [f.reshape(B * D // 128, 128) for f in flat_out]

    def perm(a):
        return a.reshape(-1, _BB // 8, 8).transpose(0, 2, 1).reshape(B)

    ids8 = jnp.stack(
        [perm(a) for a in (uid, ipid, inegid, hid, ptid, ntid, i32(r))]
        + [jnp.zeros((B,), jnp.int32)], axis=1)
    tails = [user_W[N_USR_VALID:], item_W[N_BIG_VALID:], entity_W[N_BIG_VALID:]]
    M2d = trans_M.reshape(NREL, D * D)
    return _tc_call(gathered, ids8, rel_W, M2d, tails)


# speedup vs baseline: 8.5642x; 1.0035x over previous
import functools

import jax
import jax.numpy as jnp
from jax import lax
from jax.experimental import pallas as pl
from jax.experimental.pallas import tpu as pltpu
from jax.experimental.pallas import tpu_sc as plsc

B = 16384
D = 16
N_BIG = 1000000
N_USR = 100000
NREL = 64
CF_LAMBDA = 1e-05
KG_LAMBDA = 1e-05

_NC, _NS = 2, 16
NW = _NC * _NS
BPW = B // NW
EPW = BPW * D


_CW = 16384


@functools.cache
def _make_sc_detile():
    mesh = plsc.VectorSubcoreMesh(core_axis_name="c", subcore_axis_name="s")

    SPAN_BIG = 999424 // NW
    SPAN_USR = 98304 // NW

    @functools.partial(
        pl.kernel,
        mesh=mesh,
        out_type=[
            jax.ShapeDtypeStruct((D * N_USR,), jnp.float32),
            jax.ShapeDtypeStruct((D * N_BIG,), jnp.float32),
            jax.ShapeDtypeStruct((D * N_BIG,), jnp.float32),
        ],
        scratch_types=[
            pltpu.VMEM((SPAN_BIG,), jnp.float32),
            pltpu.VMEM((SPAN_BIG,), jnp.float32),
            pltpu.VMEM((64,), jnp.float32),
            pltpu.SemaphoreType.DMA,
            pltpu.SemaphoreType.DMA,
            pltpu.SemaphoreType.DMA,
            pltpu.SemaphoreType.DMA,
        ],
    )
    def _sc_detile(userT, itemT, entityT, user_flat, item_flat, entity_flat,
                   vbuf0, vbuf1, zbuf, rsem0, rsem1, wsem0, wsem1):
        wid = lax.axis_index("s") * _NC + lax.axis_index("c")

        for i0 in range(0, 64, 16):
            zbuf[pl.ds(i0, 16)] = jnp.zeros((16,), jnp.float32)

        tasks = []
        for j in range(D):
            tasks.append((itemT, item_flat, N_BIG, j, SPAN_BIG))
            tasks.append((entityT, entity_flat, N_BIG, j, SPAN_BIG))
            tasks.append((userT, user_flat, N_USR, j, SPAN_USR))

        bufs = [vbuf0, vbuf1]
        rsems = [rsem0, rsem1]
        wsems = [wsem0, wsem1]
        pending = [None, None]
        for k, (src, dst, n, j, span) in enumerate(tasks):
            b = k % 2
            if pending[b] is not None:
                pending[b].wait()
            c0 = wid * span
            buf = bufs[b].at[pl.ds(0, span)]
            pltpu.async_copy(src.at[j, pl.ds(c0, span)], buf, rsems[b]).wait()
            pending[b] = pltpu.async_copy(
                buf, dst.at[pl.ds(j * n + c0, span)], wsems[b])
        for p in pending:
            if p is not None:
                p.wait()

        for j in range(D):
            @pl.when(wid == j)
            def _():
                pltpu.sync_copy(itemT.at[j, pl.ds(999424, 512)],
                                vbuf0.at[pl.ds(0, 512)])
                pltpu.sync_copy(vbuf0.at[pl.ds(0, 512)],
                                item_flat.at[pl.ds(j * N_BIG + 999424, 512)])
                pltpu.sync_copy(userT.at[j, pl.ds(98304, 1664)],
                                vbuf0.at[pl.ds(0, 1664)])
                pltpu.sync_copy(vbuf0.at[pl.ds(0, 1664)],
                                user_flat.at[pl.ds(j * N_USR + 98304, 1664)])
                pltpu.sync_copy(zbuf.at[pl.ds(0, 32)],
                                user_flat.at[pl.ds(j * N_USR + 99968, 32)])

            @pl.when(wid == D + j)
            def _():
                pltpu.sync_copy(entityT.at[j, pl.ds(999424, 512)],
                                vbuf0.at[pl.ds(0, 512)])
                pltpu.sync_copy(vbuf0.at[pl.ds(0, 512)],
                                entity_flat.at[pl.ds(j * N_BIG + 999424, 512)])
                pltpu.sync_copy(zbuf,
                                item_flat.at[pl.ds(j * N_BIG + 999936, 64)])
                pltpu.sync_copy(zbuf,
                                entity_flat.at[pl.ds(j * N_BIG + 999936, 64)])

    return _sc_detile


@functools.cache
def _make_sc_gather():
    mesh = plsc.VectorSubcoreMesh(core_axis_name="c", subcore_axis_name="s")

    @functools.partial(
        pl.kernel,
        mesh=mesh,
        out_type=[jax.ShapeDtypeStruct((B * D,), jnp.float32)] * 8,
        scratch_types=[
            pltpu.VMEM((EPW,), jnp.int32),
            pltpu.VMEM((EPW,), jnp.int32),
            pltpu.VMEM((EPW,), jnp.float32),
            pltpu.VMEM((EPW,), jnp.float32),
            pltpu.SemaphoreType.DMA,
            pltpu.SemaphoreType.DMA,
            pltpu.SemaphoreType.DMA,
            pltpu.SemaphoreType.DMA,
            pltpu.SemaphoreType.DMA,
        ],
    )
    def _sc_gather(user_flat, item_flat, entity_flat,
                   off_u, off_ip, off_ineg, off_h, off_pt, off_nt,
                   u_out, ip_out, ineg_out, ipk_out, inegk_out,
                   he_out, pt_out, nt_out,
                   idx0, idx1, rows0, rows1, isem0, isem1, gsem,
                   wsem0, wsem1):
        wid = lax.axis_index("s") * _NC + lax.axis_index("c")
        base = wid * EPW

        groups = [
            (off_u, [(user_flat, u_out)]),
            (off_ip, [(item_flat, ip_out), (entity_flat, ipk_out)]),
            (off_ineg, [(item_flat, ineg_out), (entity_flat, inegk_out)]),
            (off_h, [(entity_flat, he_out)]),
            (off_pt, [(entity_flat, pt_out)]),
            (off_nt, [(entity_flat, nt_out)]),
        ]
        idxs = [idx0, idx1]
        isems = [isem0, isem1]
        rows = [rows0, rows1]
        wsems = [wsem0, wsem1]

        ipend = [None, None]
        ipend[0] = pltpu.async_copy(
            groups[0][0].at[pl.ds(base, EPW)], idxs[0], isems[0])
        wpend = [None, None]
        k = 0
        for g, (off, pairs) in enumerate(groups):
            gb = g % 2
            ipend[gb].wait()
            if g + 1 < len(groups):
                nb = (g + 1) % 2
                ipend[nb] = pltpu.async_copy(
                    groups[g + 1][0].at[pl.ds(base, EPW)], idxs[nb],
                    isems[nb])
            for tab, out in pairs:
                rb = k % 2
                if wpend[rb] is not None:
                    wpend[rb].wait()
                pltpu.async_copy(tab.at[idxs[gb]], rows[rb], gsem).wait()
                wpend[rb] = pltpu.async_copy(
                    rows[rb], out.at[pl.ds(base, EPW)], wsems[rb])
                k += 1
        for p in wpend:
            if p is not None:
                p.wait()

    return _sc_gather


N_USR_VALID = 99968
N_BIG_VALID = 999936


def _patch(x, id_col, n_valid, tail_tab, width):
    oh = (id_col - n_valid == lax.broadcasted_iota(jnp.int32, (1, width), 1))
    patched = jnp.dot(oh.astype(jnp.float32), tail_tab,
                      preferred_element_type=jnp.float32)
    return jnp.where(id_col >= n_valid, patched, x)


def _tc_body(u_ref, ip_ref, ineg_ref, ipk_ref, inegk_ref,
             he_ref, pt_ref, nt_ref,
             ids_ref, relW_ref, M2d_ref,
             tailu_ref, taili_ref, taile_ref, out_ref):
    i = pl.program_id(0)

    @pl.when(i == 0)
    def _init():
        out_ref[...] = jnp.zeros((1, 1), jnp.float32)

    tailu = tailu_ref[...]
    taili = taili_ref[...]
    taile = taile_ref[...]
    ids = ids_ref[...]
    uid = ids[:, 0:1]
    ipid = ids[:, 1:2]
    inegid = ids[:, 2:3]
    hid = ids[:, 3:4]
    ptid = ids[:, 4:5]
    ntid = ids[:, 5:6]
    r_col = ids[:, 6:7]

    def unpack(ref):
        x = ref[...]
        return jnp.concatenate(
            [x[:, D * s:D * (s + 1)] for s in range(8)], axis=0)

    u = _patch(unpack(u_ref), uid, N_USR_VALID, tailu, 32)
    ip = _patch(unpack(ip_ref), ipid, N_BIG_VALID, taili, 64)
    ipk = _patch(unpack(ipk_ref), ipid, N_BIG_VALID, taile, 64)
    ineg = _patch(unpack(ineg_ref), inegid, N_BIG_VALID, taili, 64)
    inegk = _patch(unpack(inegk_ref), inegid, N_BIG_VALID, taile, 64)
    pos_cf = ip + ipk
    neg_cf = ineg + inegk
    pos_s = jnp.sum(u * pos_cf, axis=1, keepdims=True)
    neg_s = jnp.sum(u * neg_cf, axis=1, keepdims=True)
    x = pos_s - neg_s
    sig = 1.0 / (1.0 + jnp.exp(-x))
    cf_term = -jnp.log(1e-10 + sig)
    l2_cf = 0.5 * (jnp.sum(u * u) + jnp.sum(pos_cf * pos_cf)
                   + jnp.sum(neg_cf * neg_cf))

    k_row = lax.broadcasted_iota(jnp.int32, (1, NREL), 1)
    onehot = (r_col == k_row).astype(jnp.float32)
    Weff = jnp.dot(onehot, M2d_ref[...],
                   preferred_element_type=jnp.float32)
    re = jnp.dot(onehot, relW_ref[...],
                 preferred_element_type=jnp.float32)

    c1 = lax.broadcasted_iota(jnp.int32, (D, D * D), 1)
    d1 = lax.broadcasted_iota(jnp.int32, (D, D * D), 0)
    R = ((c1 // D) == d1).astype(jnp.float32)
    c2 = lax.broadcasted_iota(jnp.int32, (D * D, D), 0)
    j2 = lax.broadcasted_iota(jnp.int32, (D * D, D), 1)
    S = ((c2 % D) == j2).astype(jnp.float32)

    def proj(x16):
        xr = jnp.dot(x16, R, preferred_element_type=jnp.float32)
        return jnp.dot(xr * Weff, S, preferred_element_type=jnp.float32)

    rh = proj(_patch(unpack(he_ref), hid, N_BIG_VALID, taile, 64))
    rpt = proj(_patch(unpack(pt_ref), ptid, N_BIG_VALID, taile, 64))
    rnt = proj(_patch(unpack(nt_ref), ntid, N_BIG_VALID, taile, 64))

    def normz(v):
        n = jnp.sqrt(jnp.sum(v * v, axis=1, keepdims=True))
        return v / jnp.maximum(n, 1e-12)

    re_n = normz(re)
    rh_n = normz(rh)
    rpt_n = normz(rpt)
    rnt_n = normz(rnt)

    dpos = rh_n + re_n - rpt_n
    dneg = rh_n + re_n - rnt_n
    pos_sc = jnp.sqrt(jnp.sum(dpos * dpos, axis=1, keepdims=True))
    neg_sc = jnp.sqrt(jnp.sum(dneg * dneg, axis=1, keepdims=True))
    kg_term = jnp.maximum(pos_sc - neg_sc + 1.0, 0.0)
    l2_kg = 0.5 * (jnp.sum(rh_n * rh_n) + jnp.sum(re_n * re_n)
                   + jnp.sum(rpt_n * rpt_n) + jnp.sum(rnt_n * rnt_n))

    block_total = (jnp.sum(cf_term) + CF_LAMBDA * l2_cf
                   + jnp.sum(kg_term) + KG_LAMBDA * l2_kg)
    out_ref[...] += jnp.reshape(block_total * (1.0 / B), (1, 1))


_BB = 4096


def _tc_call(gathered, ids8, rel_W, M2d, tails, interpret=False):
    row_spec = pl.BlockSpec((_BB // 8, 128), lambda i: (i, 0))
    out = pl.pallas_call(
        _tc_body,
        grid=(B // _BB,),
        in_specs=[row_spec] * 8 + [
            pl.BlockSpec((_BB, 8), lambda i: (i, 0)),
        ] + [
            pl.BlockSpec((NREL, D), lambda i: (0, 0)),
            pl.BlockSpec((NREL, D * D), lambda i: (0, 0)),
            pl.BlockSpec((32, D), lambda i: (0, 0)),
            pl.BlockSpec((64, D), lambda i: (0, 0)),
            pl.BlockSpec((64, D), lambda i: (0, 0)),
        ],
        out_specs=pl.BlockSpec((1, 1), lambda i: (0, 0)),
        out_shape=jax.ShapeDtypeStruct((1, 1), jnp.float32),
        interpret=interpret,
    )(*gathered, ids8, rel_W, M2d, *tails)
    return out[0, 0]


def _offsets(idx, n_rows):
    k = jnp.arange(B * D, dtype=jnp.int32)
    return (k % D) * n_rows + jnp.repeat(idx, D)


def kernel(user_W, item_W, entity_W, rel_W, trans_M,
           user_ids, item_pos_ids, item_neg_ids, h, r, pos_t, neg_t,
           is_train=1):
    i32 = lambda a: a.astype(jnp.int32)
    uid, ipid, inegid = i32(user_ids), i32(item_pos_ids), i32(item_neg_ids)
    hid, ptid, ntid = i32(h), i32(pos_t), i32(neg_t)

    user_flat, item_flat, entity_flat = _make_sc_detile()(
        user_W.T, item_W.T, entity_W.T)

    offs = [_offsets(uid, N_USR)] + [
        _offsets(a, N_BIG) for a in (ipid, inegid, hid, ptid, ntid)]

    flat_out = _make_sc_gather()(user_flat, item_flat, entity_flat, *offs)
    gathered = [f.reshape(B * D // 128, 128) for f in flat_out]

    def perm(a):
        return a.reshape(-1, _BB // 8, 8).transpose(0, 2, 1).reshape(B)

    ids8 = jnp.stack(
        [perm(a) for a in (uid, ipid, inegid, hid, ptid, ntid, i32(r))]
        + [jnp.zeros((B,), jnp.int32)], axis=1)
    tails = [user_W[N_USR_VALID:], item_W[N_BIG_VALID:], entity_W[N_BIG_VALID:]]
    M2d = trans_M.reshape(NREL, D * D)
    return _tc_call(gathered, ids8, rel_W, M2d, tails)


# speedup vs baseline: 8.8583x; 1.0343x over previous
import functools

import jax
import jax.numpy as jnp
from jax import lax
from jax.experimental import pallas as pl
from jax.experimental.pallas import tpu as pltpu
from jax.experimental.pallas import tpu_sc as plsc

B = 16384
D = 16
N_BIG = 1000000
N_USR = 100000
NREL = 64
CF_LAMBDA = 1e-05
KG_LAMBDA = 1e-05

_NC, _NS = 2, 16
NW = _NC * _NS
BPW = B // NW
EPW = BPW * D


_CW = 16384


@functools.cache
def _make_sc_detile():
    mesh = plsc.VectorSubcoreMesh(core_axis_name="c", subcore_axis_name="s")

    SPAN_BIG = 999424 // NW
    SPAN_USR = 98304 // NW

    @functools.partial(
        pl.kernel,
        mesh=mesh,
        out_type=[
            jax.ShapeDtypeStruct((D * N_USR,), jnp.float32),
            jax.ShapeDtypeStruct((D * N_BIG,), jnp.float32),
            jax.ShapeDtypeStruct((D * N_BIG,), jnp.float32),
        ],
        scratch_types=[
            pltpu.VMEM((SPAN_BIG,), jnp.float32),
            pltpu.VMEM((SPAN_BIG,), jnp.float32),
            pltpu.VMEM((64,), jnp.float32),
            pltpu.SemaphoreType.DMA,
            pltpu.SemaphoreType.DMA,
            pltpu.SemaphoreType.DMA,
            pltpu.SemaphoreType.DMA,
        ],
    )
    def _sc_detile(userT, itemT, entityT, user_flat, item_flat, entity_flat,
                   vbuf0, vbuf1, zbuf, rsem0, rsem1, wsem0, wsem1):
        wid = lax.axis_index("s") * _NC + lax.axis_index("c")

        for i0 in range(0, 64, 16):
            zbuf[pl.ds(i0, 16)] = jnp.zeros((16,), jnp.float32)

        tasks = []
        for j in range(D):
            tasks.append((itemT, item_flat, N_BIG, j, SPAN_BIG))
            tasks.append((entityT, entity_flat, N_BIG, j, SPAN_BIG))
            tasks.append((userT, user_flat, N_USR, j, SPAN_USR))

        bufs = [vbuf0, vbuf1]
        rsems = [rsem0, rsem1]
        wsems = [wsem0, wsem1]
        pending = [None, None]
        for k, (src, dst, n, j, span) in enumerate(tasks):
            b = k % 2
            if pending[b] is not None:
                pending[b].wait()
            c0 = wid * span
            buf = bufs[b].at[pl.ds(0, span)]
            pltpu.async_copy(src.at[j, pl.ds(c0, span)], buf, rsems[b]).wait()
            pending[b] = pltpu.async_copy(
                buf, dst.at[pl.ds(j * n + c0, span)], wsems[b])
        for p in pending:
            if p is not None:
                p.wait()

        for j in range(D):
            @pl.when(wid == j)
            def _():
                pltpu.sync_copy(itemT.at[j, pl.ds(999424, 512)],
                                vbuf0.at[pl.ds(0, 512)])
                pltpu.sync_copy(vbuf0.at[pl.ds(0, 512)],
                                item_flat.at[pl.ds(j * N_BIG + 999424, 512)])
                pltpu.sync_copy(userT.at[j, pl.ds(98304, 1664)],
                                vbuf0.at[pl.ds(0, 1664)])
                pltpu.sync_copy(vbuf0.at[pl.ds(0, 1664)],
                                user_flat.at[pl.ds(j * N_USR + 98304, 1664)])
                pltpu.sync_copy(zbuf.at[pl.ds(0, 32)],
                                user_flat.at[pl.ds(j * N_USR + 99968, 32)])

            @pl.when(wid == D + j)
            def _():
                pltpu.sync_copy(entityT.at[j, pl.ds(999424, 512)],
                                vbuf0.at[pl.ds(0, 512)])
                pltpu.sync_copy(vbuf0.at[pl.ds(0, 512)],
                                entity_flat.at[pl.ds(j * N_BIG + 999424, 512)])
                pltpu.sync_copy(zbuf,
                                item_flat.at[pl.ds(j * N_BIG + 999936, 64)])
                pltpu.sync_copy(zbuf,
                                entity_flat.at[pl.ds(j * N_BIG + 999936, 64)])

    return _sc_detile


HALF = B * D // 2


@functools.cache
def _make_sc_gather(half):
    mesh = plsc.VectorSubcoreMesh(core_axis_name="c", subcore_axis_name="s")

    @functools.partial(
        pl.kernel,
        mesh=mesh,
        out_type=[jax.ShapeDtypeStruct((HALF,), jnp.float32)] * 8,
        scratch_types=[
            pltpu.VMEM((EPW // 2,), jnp.int32),
            pltpu.VMEM((EPW // 2,), jnp.int32),
            pltpu.VMEM((EPW // 2,), jnp.float32),
            pltpu.VMEM((EPW // 2,), jnp.float32),
            pltpu.SemaphoreType.DMA,
            pltpu.SemaphoreType.DMA,
            pltpu.SemaphoreType.DMA,
            pltpu.SemaphoreType.DMA,
            pltpu.SemaphoreType.DMA,
        ],
    )
    def _sc_gather(user_flat, item_flat, entity_flat,
                   off_u, off_ip, off_ineg, off_h, off_pt, off_nt,
                   u_out, ip_out, ineg_out, ipk_out, inegk_out,
                   he_out, pt_out, nt_out,
                   idx0, idx1, rows0, rows1, isem0, isem1, gsem,
                   wsem0, wsem1):
        wid = lax.axis_index("s") * _NC + lax.axis_index("c")
        src_base = half * HALF + wid * (EPW // 2)
        base = wid * (EPW // 2)

        groups = [
            (off_u, [(user_flat, u_out)]),
            (off_ip, [(item_flat, ip_out), (entity_flat, ipk_out)]),
            (off_ineg, [(item_flat, ineg_out), (entity_flat, inegk_out)]),
            (off_h, [(entity_flat, he_out)]),
            (off_pt, [(entity_flat, pt_out)]),
            (off_nt, [(entity_flat, nt_out)]),
        ]
        idxs = [idx0, idx1]
        isems = [isem0, isem1]
        rows = [rows0, rows1]
        wsems = [wsem0, wsem1]

        ipend = [None, None]
        ipend[0] = pltpu.async_copy(
            groups[0][0].at[pl.ds(src_base, EPW // 2)], idxs[0], isems[0])
        wpend = [None, None]
        k = 0
        for g, (off, pairs) in enumerate(groups):
            gb = g % 2
            ipend[gb].wait()
            if g + 1 < len(groups):
                nb = (g + 1) % 2
                ipend[nb] = pltpu.async_copy(
                    groups[g + 1][0].at[pl.ds(src_base, EPW // 2)], idxs[nb],
                    isems[nb])
            for tab, out in pairs:
                rb = k % 2
                if wpend[rb] is not None:
                    wpend[rb].wait()
                pltpu.async_copy(tab.at[idxs[gb]], rows[rb], gsem).wait()
                wpend[rb] = pltpu.async_copy(
                    rows[rb], out.at[pl.ds(base, EPW // 2)], wsems[rb])
                k += 1
        for p in wpend:
            if p is not None:
                p.wait()

    return _sc_gather


N_USR_VALID = 99968
N_BIG_VALID = 999936


def _patch(x, id_col, n_valid, tail_tab, width):
    oh = (id_col - n_valid == lax.broadcasted_iota(jnp.int32, (1, width), 1))
    patched = jnp.dot(oh.astype(jnp.float32), tail_tab,
                      preferred_element_type=jnp.float32)
    return jnp.where(id_col >= n_valid, patched, x)


def _tc_body(u_ref, ip_ref, ineg_ref, ipk_ref, inegk_ref,
             he_ref, pt_ref, nt_ref,
             ids_ref, relW_ref, M2d_ref,
             tailu_ref, taili_ref, taile_ref, out_ref):
    i = pl.program_id(0)

    @pl.when(i == 0)
    def _init():
        out_ref[...] = jnp.zeros((1, 1), jnp.float32)

    tailu = tailu_ref[...]
    taili = taili_ref[...]
    taile = taile_ref[...]
    ids = ids_ref[...]
    uid = ids[:, 0:1]
    ipid = ids[:, 1:2]
    inegid = ids[:, 2:3]
    hid = ids[:, 3:4]
    ptid = ids[:, 4:5]
    ntid = ids[:, 5:6]
    r_col = ids[:, 6:7]

    def unpack(ref):
        x = ref[...]
        return jnp.concatenate(
            [x[:, D * s:D * (s + 1)] for s in range(8)], axis=0)

    u = _patch(unpack(u_ref), uid, N_USR_VALID, tailu, 32)
    ip = _patch(unpack(ip_ref), ipid, N_BIG_VALID, taili, 64)
    ipk = _patch(unpack(ipk_ref), ipid, N_BIG_VALID, taile, 64)
    ineg = _patch(unpack(ineg_ref), inegid, N_BIG_VALID, taili, 64)
    inegk = _patch(unpack(inegk_ref), inegid, N_BIG_VALID, taile, 64)
    pos_cf = ip + ipk
    neg_cf = ineg + inegk
    pos_s = jnp.sum(u * pos_cf, axis=1, keepdims=True)
    neg_s = jnp.sum(u * neg_cf, axis=1, keepdims=True)
    x = pos_s - neg_s
    sig = 1.0 / (1.0 + jnp.exp(-x))
    cf_term = -jnp.log(1e-10 + sig)
    l2_cf = 0.5 * (jnp.sum(u * u) + jnp.sum(pos_cf * pos_cf)
                   + jnp.sum(neg_cf * neg_cf))

    k_row = lax.broadcasted_iota(jnp.int32, (1, NREL), 1)
    onehot = (r_col == k_row).astype(jnp.float32)
    Weff = jnp.dot(onehot, M2d_ref[...],
                   preferred_element_type=jnp.float32)
    re = jnp.dot(onehot, relW_ref[...],
                 preferred_element_type=jnp.float32)

    c1 = lax.broadcasted_iota(jnp.int32, (D, D * D), 1)
    d1 = lax.broadcasted_iota(jnp.int32, (D, D * D), 0)
    R = ((c1 // D) == d1).astype(jnp.float32)
    c2 = lax.broadcasted_iota(jnp.int32, (D * D, D), 0)
    j2 = lax.broadcasted_iota(jnp.int32, (D * D, D), 1)
    S = ((c2 % D) == j2).astype(jnp.float32)

    def proj(x16):
        xr = jnp.dot(x16, R, preferred_element_type=jnp.float32)
        return jnp.dot(xr * Weff, S, preferred_element_type=jnp.float32)

    rh = proj(_patch(unpack(he_ref), hid, N_BIG_VALID, taile, 64))
    rpt = proj(_patch(unpack(pt_ref), ptid, N_BIG_VALID, taile, 64))
    rnt = proj(_patch(unpack(nt_ref), ntid, N_BIG_VALID, taile, 64))

    def normz(v):
        n = jnp.sqrt(jnp.sum(v * v, axis=1, keepdims=True))
        return v / jnp.maximum(n, 1e-12)

    re_n = normz(re)
    rh_n = normz(rh)
    rpt_n = normz(rpt)
    rnt_n = normz(rnt)

    dpos = rh_n + re_n - rpt_n
    dneg = rh_n + re_n - rnt_n
    pos_sc = jnp.sqrt(jnp.sum(dpos * dpos, axis=1, keepdims=True))
    neg_sc = jnp.sqrt(jnp.sum(dneg * dneg, axis=1, keepdims=True))
    kg_term = jnp.maximum(pos_sc - neg_sc + 1.0, 0.0)
    l2_kg = 0.5 * (jnp.sum(rh_n * rh_n) + jnp.sum(re_n * re_n)
                   + jnp.sum(rpt_n * rpt_n) + jnp.sum(rnt_n * rnt_n))

    block_total = (jnp.sum(cf_term) + CF_LAMBDA * l2_cf
                   + jnp.sum(kg_term) + KG_LAMBDA * l2_kg)
    out_ref[...] += jnp.reshape(block_total * (1.0 / B), (1, 1))


_BB = 4096


def _tc_call(gathered, ids8, rel_W, M2d, tails, interpret=False):
    row_spec = pl.BlockSpec((_BB // 8, 128), lambda i: (i, 0))
    out = pl.pallas_call(
        _tc_body,
        grid=(B // 2 // _BB,),
        in_specs=[row_spec] * 8 + [
            pl.BlockSpec((_BB, 8), lambda i: (i, 0)),
        ] + [
            pl.BlockSpec((NREL, D), lambda i: (0, 0)),
            pl.BlockSpec((NREL, D * D), lambda i: (0, 0)),
            pl.BlockSpec((32, D), lambda i: (0, 0)),
            pl.BlockSpec((64, D), lambda i: (0, 0)),
            pl.BlockSpec((64, D), lambda i: (0, 0)),
        ],
        out_specs=pl.BlockSpec((1, 1), lambda i: (0, 0)),
        out_shape=jax.ShapeDtypeStruct((1, 1), jnp.float32),
        interpret=interpret,
    )(*gathered, ids8, rel_W, M2d, *tails)
    return out[0, 0]


def _offsets(idx, n_rows):
    k = jnp.arange(B * D, dtype=jnp.int32)
    return (k % D) * n_rows + jnp.repeat(idx, D)


def kernel(user_W, item_W, entity_W, rel_W, trans_M,
           user_ids, item_pos_ids, item_neg_ids, h, r, pos_t, neg_t,
           is_train=1):
    i32 = lambda a: a.astype(jnp.int32)
    uid, ipid, inegid = i32(user_ids), i32(item_pos_ids), i32(item_neg_ids)
    hid, ptid, ntid = i32(h), i32(pos_t), i32(neg_t)

    user_flat, item_flat, entity_flat = _make_sc_detile()(
        user_W.T, item_W.T, entity_W.T)

    offs = [_offsets(uid, N_USR)] + [
        _offsets(a, N_BIG) for a in (ipid, inegid, hid, ptid, ntid)]

    def perm(a):
        return a.reshape(-1, _BB // 8, 8).transpose(0, 2, 1).reshape(B)

    ids8 = jnp.stack(
        [perm(a) for a in (uid, ipid, inegid, hid, ptid, ntid, i32(r))]
        + [jnp.zeros((B,), jnp.int32)], axis=1)
    tails = [user_W[N_USR_VALID:], item_W[N_BIG_VALID:], entity_W[N_BIG_VALID:]]
    M2d = trans_M.reshape(NREL, D * D)

    total = None
    for half in range(2):
        flat_out = _make_sc_gather(half)(
            user_flat, item_flat, entity_flat, *offs)
        gathered = [f.reshape(HALF // 128, 128) for f in flat_out]
        ids8_h = ids8[half * (B // 2):(half + 1) * (B // 2)]
        part = _tc_call(gathered, ids8_h, rel_W, M2d, tails)
        total = part if total is None else total + part
    return total
